# Initial kernel scaffold; baseline (speedup 1.0000x reference)
import jax
import jax.numpy as jnp
from jax import lax
from jax.experimental import pallas as pl
from jax.experimental.pallas import tpu as pltpu
from jax.experimental.pallas import tpu_sc as plsc

_N = 10000
_E = 320000
_D = 128
_H = 64
_NL = 3
_G = 64
_NC = 2
_NS = 16
_NPAD = 10240
_EPAD = 327680
_EPS = _EPAD // _NS
_EK = 128
_BN = 1000
_NSTEPS = _N // _BN


def _sc_edge_body(h_hbm, src_hbm, dst_hbm, scores_hbm, p_out, q_out,
                  scores_v, src_v, srcg_v, dst_v, alpha_v, rows_v, wrows_v,
                  zbuf, p_acc, q_acc, sem):
    cid = lax.axis_index("c")
    sid = lax.axis_index("s")

    def _zrow(j, c):
        for t in range(_H // 16):
            zbuf[j, pl.ds(t * 16, 16)] = jnp.zeros((16,), jnp.float32)
        return c
    lax.fori_loop(0, _EK, _zrow, 0)

    rows_per_tile = _NPAD // _NS
    def _zacc(ci, c):
        base = sid * rows_per_tile + ci * _EK
        pltpu.sync_copy(zbuf, p_acc.at[pl.ds(base, _EK)])
        pltpu.sync_copy(zbuf, q_acc.at[pl.ds(base, _EK)])
        return c
    lax.fori_loop(0, rows_per_tile // _EK, _zacc, 0)

    pltpu.sync_copy(scores_hbm, scores_v)
    plsc.subcore_barrier()

    ebase = sid * _EPS
    toff = cid * _N

    def _chunk(ci, c):
        b = ebase + ci * _EK
        pltpu.sync_copy(src_hbm.at[pl.ds(b, _EK)], src_v)
        pltpu.sync_copy(dst_hbm.at[pl.ds(b, _EK)], dst_v)
        for g in range(_EK // 16):
            sl = pl.ds(g * 16, 16)
            si = src_v[sl]
            di = dst_v[sl]
            ss = plsc.load_gather(scores_v, [si])
            sd = plsc.load_gather(scores_v, [di])
            alpha_v[sl] = 1.0 / (1.0 + jnp.exp(sd - ss))
            srcg_v[sl] = si + toff
        pltpu.async_copy(h_hbm.at[srcg_v], rows_v, sem).wait()

        def _wr(j, c2):
            a = alpha_v[j]
            for t in range(_H // 16):
                sl2 = pl.ds(t * 16, 16)
                wrows_v[j, sl2] = rows_v[j, sl2] * a
            return c2
        lax.fori_loop(0, _EK, _wr, 0)

        pltpu.sync_copy(wrows_v, p_acc.at[dst_v], add=True)
        pltpu.sync_copy(rows_v, q_acc.at[dst_v], add=True)
        return c
    lax.fori_loop(0, _EPS // _EK, _chunk, 0)

    plsc.subcore_barrier()

    obase = sid * rows_per_tile
    nvalid = _N - (_NS - 1) * rows_per_tile

    @pl.when(sid < _NS - 1)
    def _():
        pltpu.sync_copy(p_acc.at[pl.ds(obase, rows_per_tile)],
                        p_out.at[cid, pl.ds(obase, rows_per_tile)])
        pltpu.sync_copy(q_acc.at[pl.ds(obase, rows_per_tile)],
                        q_out.at[cid, pl.ds(obase, rows_per_tile)])

    @pl.when(sid == _NS - 1)
    def _():
        pltpu.sync_copy(p_acc.at[pl.ds(obase, nvalid)],
                        p_out.at[cid, pl.ds(obase, nvalid)])
        pltpu.sync_copy(q_acc.at[pl.ds(obase, nvalid)],
                        q_out.at[cid, pl.ds(obase, nvalid)])


def _sc_edge(hsplit_flat, src_pad, dst_pad, scores_pad):
    mesh = plsc.VectorSubcoreMesh(core_axis_name="c", subcore_axis_name="s")
    f32 = jnp.float32
    run = pl.kernel(
        _sc_edge_body,
        out_type=(jax.ShapeDtypeStruct((_NC, _N, _H), f32),
                  jax.ShapeDtypeStruct((_NC, _N, _H), f32)),
        mesh=mesh,
        scratch_types=[
            pltpu.VMEM((_NPAD,), f32),
            pltpu.VMEM((_EK,), jnp.int32),
            pltpu.VMEM((_EK,), jnp.int32),
            pltpu.VMEM((_EK,), jnp.int32),
            pltpu.VMEM((_EK,), f32),
            pltpu.VMEM((_EK, _H), f32),
            pltpu.VMEM((_EK, _H), f32),
            pltpu.VMEM((_EK, _H), f32),
            pltpu.VMEM_SHARED((_NPAD, _H), f32),
            pltpu.VMEM_SHARED((_NPAD, _H), f32),
            pltpu.SemaphoreType.DMA,
        ],
    )
    return run(hsplit_flat, src_pad, dst_pad, scores_pad)


def _fuse_body(self_w, fwd_w, bwd_w, comb_w, self_b, comb_b,
               ws_o, wd_o, wb_o, bias_o):
    c1 = comb_w[0, :_D, :]
    c2 = comb_w[0, _D:2 * _D, :]
    c3 = comb_w[0, 2 * _D:, :]
    f32 = jnp.float32
    ws_o[0] = jnp.dot(self_w[0], c1, preferred_element_type=f32)
    wf = jnp.dot(fwd_w[0], c2, preferred_element_type=f32)
    wb = jnp.dot(bwd_w[0], c3, preferred_element_type=f32)
    wd_o[0] = wf - wb
    wb_o[0] = wb
    bias_o[0, 0] = jnp.dot(self_b[0, 0], c1, preferred_element_type=f32) \
        + comb_b[0, 0]


def _fuse_weights(self_W, fwd_W, bwd_W, comb_W, self_b, comb_b):
    f32 = jnp.float32
    w_spec = pl.BlockSpec((1, _D, _D), lambda l: (l, 0, 0))
    b_spec = pl.BlockSpec((1, 1, _D), lambda l: (l, 0, 0))
    return pl.pallas_call(
        _fuse_body,
        grid=(_NL,),
        in_specs=[w_spec, w_spec, w_spec,
                  pl.BlockSpec((1, 3 * _D, _D), lambda l: (l, 0, 0)),
                  b_spec, b_spec],
        out_specs=[w_spec, w_spec, w_spec, b_spec],
        out_shape=[jax.ShapeDtypeStruct((_NL, _D, _D), f32),
                   jax.ShapeDtypeStruct((_NL, _D, _D), f32),
                   jax.ShapeDtypeStruct((_NL, _D, _D), f32),
                   jax.ShapeDtypeStruct((_NL, 1, _D), f32)],
    )(self_W, fwd_W, bwd_W, comb_W,
      self_b.reshape(_NL, 1, _D), comb_b.reshape(_NL, 1, _D))


def _prologue_body(x_ref, w_ref, b_ref, sw_ref, hs_o, sc_o):
    f32 = jnp.float32
    h = jnp.dot(x_ref[...], w_ref[...], preferred_element_type=f32)
    h = jax.nn.relu(h + b_ref[0])
    hs_o[0] = h[:, :_H]
    hs_o[1] = h[:, _H:]
    sc_o[0] = jnp.dot(h, sw_ref[0, 0], preferred_element_type=f32)


def _prologue(x, emb_W, emb_b, sw0):
    f32 = jnp.float32
    return pl.pallas_call(
        _prologue_body,
        grid=(_NSTEPS,),
        in_specs=[pl.BlockSpec((_BN, _D), lambda i: (i, 0)),
                  pl.BlockSpec((_D, _D), lambda i: (0, 0)),
                  pl.BlockSpec((1, _D), lambda i: (0, 0)),
                  pl.BlockSpec((1, 1, _D), lambda i: (0, 0, 0))],
        out_specs=[pl.BlockSpec((2, _BN, _H), lambda i: (0, i, 0)),
                   pl.BlockSpec((1, _BN), lambda i: (0, i))],
        out_shape=[jax.ShapeDtypeStruct((2, _N, _H), f32),
                   jax.ShapeDtypeStruct((1, _N), f32)],
    )(x, emb_W, emb_b.reshape(1, _D), sw0)


def _ln_relu_res(acc, h, g_ref, b_ref):
    mu = jnp.mean(acc, axis=-1, keepdims=True)
    var = jnp.mean((acc - mu) ** 2, axis=-1, keepdims=True)
    nrm = (acc - mu) / jnp.sqrt(var + 1e-5) * g_ref[0] + b_ref[0]
    return jax.nn.relu(nrm) + h


def _layer_body(hl, hr, pl_r, pr_r, ql, qr, ws, wd, wb, bias, g_ref, b_ref,
                sw_ref, hs_o, sc_o):
    f32 = jnp.float32
    h = jnp.concatenate([hl[0], hr[0]], axis=-1)
    p = jnp.concatenate([pl_r[0], pr_r[0]], axis=-1)
    q = jnp.concatenate([ql[0], qr[0]], axis=-1)
    acc = (jnp.dot(h, ws[...], preferred_element_type=f32)
           + jnp.dot(p, wd[...], preferred_element_type=f32)
           + jnp.dot(q, wb[...], preferred_element_type=f32)
           + bias[0])
    out = _ln_relu_res(acc, h, g_ref, b_ref)
    hs_o[0] = out[:, :_H]
    hs_o[1] = out[:, _H:]
    sc_o[0] = jnp.dot(out, sw_ref[0, 0], preferred_element_type=f32)


def _layer(hs, p2, q2, ws, wd, wb, bias, ln_g1, ln_b1, sw_next):
    f32 = jnp.float32
    half = lambda j: pl.BlockSpec((1, _BN, _H), lambda i, j=j: (j, i, 0))
    wspec = pl.BlockSpec((_D, _D), lambda i: (0, 0))
    vspec = pl.BlockSpec((1, _D), lambda i: (0, 0))
    return pl.pallas_call(
        _layer_body,
        grid=(_NSTEPS,),
        in_specs=[half(0), half(1), half(0), half(1), half(0), half(1),
                  wspec, wspec, wspec, vspec, vspec, vspec,
                  pl.BlockSpec((1, 1, _D), lambda i: (0, 0, 0))],
        out_specs=[pl.BlockSpec((2, _BN, _H), lambda i: (0, i, 0)),
                   pl.BlockSpec((1, _BN), lambda i: (0, i))],
        out_shape=[jax.ShapeDtypeStruct((2, _N, _H), f32),
                   jax.ShapeDtypeStruct((1, _N), f32)],
    )(hs, hs, p2, p2, q2, q2, ws, wd, wb, bias, ln_g1, ln_b1, sw_next)


def _final_body(hl, hr, pl_r, pr_r, ql, qr, ws, wd, wb, bias, g_ref, b_ref,
                batch_ref, pooled_o, sum_acc, cnt_acc):
    f32 = jnp.float32
    i = pl.program_id(0)
    h = jnp.concatenate([hl[0], hr[0]], axis=-1)
    p = jnp.concatenate([pl_r[0], pr_r[0]], axis=-1)
    q = jnp.concatenate([ql[0], qr[0]], axis=-1)
    acc = (jnp.dot(h, ws[...], preferred_element_type=f32)
           + jnp.dot(p, wd[...], preferred_element_type=f32)
           + jnp.dot(q, wb[...], preferred_element_type=f32)
           + bias[0])
    out = _ln_relu_res(acc, h, g_ref, b_ref)

    @pl.when(i == 0)
    def _():
        sum_acc[...] = jnp.zeros((_G, _D), f32)
        cnt_acc[...] = jnp.zeros((_G, 1), f32)

    gids = lax.broadcasted_iota(jnp.int32, (_G, _BN), 0)
    oh = (gids == batch_ref[0][None, :]).astype(f32)
    sum_acc[...] += jnp.dot(oh, out, preferred_element_type=f32)
    cnt_acc[...] += jnp.sum(oh, axis=1, keepdims=True)

    @pl.when(i == _NSTEPS - 1)
    def _():
        pooled_o[...] = sum_acc[...] / jnp.maximum(cnt_acc[...], 1.0)


def _final_layer(hs, p2, q2, ws, wd, wb, bias, ln_g1, ln_b1, batch2d):
    f32 = jnp.float32
    half = lambda j: pl.BlockSpec((1, _BN, _H), lambda i, j=j: (j, i, 0))
    wspec = pl.BlockSpec((_D, _D), lambda i: (0, 0))
    vspec = pl.BlockSpec((1, _D), lambda i: (0, 0))
    return pl.pallas_call(
        _final_body,
        grid=(_NSTEPS,),
        in_specs=[half(0), half(1), half(0), half(1), half(0), half(1),
                  wspec, wspec, wspec, vspec, vspec,
                  pl.BlockSpec((1, _BN), lambda i: (0, i))],
        out_specs=pl.BlockSpec((_G, _D), lambda i: (0, 0)),
        out_shape=jax.ShapeDtypeStruct((_G, _D), f32),
        scratch_shapes=[pltpu.VMEM((_G, _D), f32),
                        pltpu.VMEM((_G, 1), f32)],
    )(hs, hs, p2, p2, q2, q2, ws, wd, wb, bias, ln_g1, ln_b1, batch2d)


def kernel(x, edge_index, batch, emb_W, emb_b, score_W, score_b, fwd_W,
           bwd_W, self_W, self_b, comb_W, comb_b, ln_g, ln_b):
    src = edge_index[0].astype(jnp.int32)
    dst = edge_index[1].astype(jnp.int32)
    npad = _EPAD - _E
    src_pad = jnp.concatenate([src, jnp.zeros((npad,), jnp.int32)])
    dst_pad = jnp.concatenate([dst, jnp.full((npad,), _N, jnp.int32)])
    batch2d = batch.astype(jnp.int32).reshape(1, _N)
    sw = score_W.reshape(_NL, 1, 1, _D)

    ws_s, wd_s, wb_s, bias_s = _fuse_weights(self_W, fwd_W, bwd_W, comb_W,
                                             self_b, comb_b)

    hs, sc = _prologue(x, emb_W, emb_b, sw[0])
    for l in range(_NL):
        scores_pad = jnp.pad(sc[0], (0, _NPAD - _N))
        p2, q2 = _sc_edge(hs.reshape(_NC * _N, _H), src_pad, dst_pad,
                          scores_pad)
        args = (hs, p2, q2, ws_s[l], wd_s[l], wb_s[l], bias_s[l],
                ln_g[l].reshape(1, _D), ln_b[l].reshape(1, _D))
        if l < _NL - 1:
            hs, sc = _layer(*args, sw[l + 1])
        else:
            pooled = _final_layer(*args, batch2d)
    return (pooled, 0)


# speedup vs baseline: 9.0695x; 9.0695x over previous
import jax
import jax.numpy as jnp
from jax import lax
from jax.experimental import pallas as pl
from jax.experimental.pallas import tpu as pltpu
from jax.experimental.pallas import tpu_sc as plsc

_N = 10000
_E = 320000
_D = 128
_H = 64
_NL = 3
_G = 64
_NC = 2
_NS = 16
_NPAD = 10240
_EPAD = 327680
_EPS = _EPAD // _NS
_EK = 128
_BN = 1000
_NSTEPS = _N // _BN


def _sc_edge_body(h_hbm, src_hbm, dst_hbm, scores_hbm, p_out, q_out,
                  scores_v, src_v, srcg_v, dst_v, alpha_v, rows_v, wrows_v,
                  zbuf, p_acc, q_acc, sem):
    cid = lax.axis_index("c")
    sid = lax.axis_index("s")

    def _zrow(j, c):
        for t in range(_H // 16):
            zbuf[j, pl.ds(t * 16, 16)] = jnp.zeros((16,), jnp.float32)
        return c
    lax.fori_loop(0, _EK, _zrow, 0)

    rows_per_tile = _NPAD // _NS
    def _zacc(ci, c):
        base = sid * rows_per_tile + ci * _EK
        pltpu.sync_copy(zbuf, p_acc.at[pl.ds(base, _EK)])
        pltpu.sync_copy(zbuf, q_acc.at[pl.ds(base, _EK)])
        return c
    lax.fori_loop(0, rows_per_tile // _EK, _zacc, 0)

    pltpu.sync_copy(scores_hbm, scores_v)
    plsc.subcore_barrier()

    ebase = sid * _EPS
    toff = cid * _N

    def _chunk(ci, c):
        b = ebase + ci * _EK
        pltpu.sync_copy(src_hbm.at[pl.ds(b, _EK)], src_v)
        pltpu.sync_copy(dst_hbm.at[pl.ds(b, _EK)], dst_v)
        for g in range(_EK // 16):
            sl = pl.ds(g * 16, 16)
            si = src_v[sl]
            di = dst_v[sl]
            ss = plsc.load_gather(scores_v, [si])
            sd = plsc.load_gather(scores_v, [di])
            alpha_v[sl] = 1.0 / (1.0 + jnp.exp(sd - ss))
            srcg_v[sl] = si + toff
        pltpu.async_copy(h_hbm.at[srcg_v], rows_v, sem).wait()

        def _wr(gi, c2):
            av = alpha_v[pl.ds(gi * 16, 16)]
            for e in range(16):
                a = av[e]
                j = gi * 16 + e
                for t in range(_H // 16):
                    sl2 = pl.ds(t * 16, 16)
                    wrows_v[j, sl2] = rows_v[j, sl2] * a
            return c2
        lax.fori_loop(0, _EK // 16, _wr, 0)

        pltpu.sync_copy(wrows_v, p_acc.at[dst_v], add=True)
        pltpu.sync_copy(rows_v, q_acc.at[dst_v], add=True)
        return c
    lax.fori_loop(0, _EPS // _EK, _chunk, 0)

    plsc.subcore_barrier()

    obase = sid * rows_per_tile
    nvalid = _N - (_NS - 1) * rows_per_tile

    @pl.when(sid < _NS - 1)
    def _():
        pltpu.sync_copy(p_acc.at[pl.ds(obase, rows_per_tile)],
                        p_out.at[cid, pl.ds(obase, rows_per_tile)])
        pltpu.sync_copy(q_acc.at[pl.ds(obase, rows_per_tile)],
                        q_out.at[cid, pl.ds(obase, rows_per_tile)])

    @pl.when(sid == _NS - 1)
    def _():
        pltpu.sync_copy(p_acc.at[pl.ds(obase, nvalid)],
                        p_out.at[cid, pl.ds(obase, nvalid)])
        pltpu.sync_copy(q_acc.at[pl.ds(obase, nvalid)],
                        q_out.at[cid, pl.ds(obase, nvalid)])


def _sc_edge(hsplit_flat, src_pad, dst_pad, scores_pad):
    mesh = plsc.VectorSubcoreMesh(core_axis_name="c", subcore_axis_name="s")
    f32 = jnp.float32
    run = pl.kernel(
        _sc_edge_body,
        out_type=(jax.ShapeDtypeStruct((_NC, _N, _H), f32),
                  jax.ShapeDtypeStruct((_NC, _N, _H), f32)),
        mesh=mesh,
        scratch_types=[
            pltpu.VMEM((_NPAD,), f32),
            pltpu.VMEM((_EK,), jnp.int32),
            pltpu.VMEM((_EK,), jnp.int32),
            pltpu.VMEM((_EK,), jnp.int32),
            pltpu.VMEM((_EK,), f32),
            pltpu.VMEM((_EK, _H), f32),
            pltpu.VMEM((_EK, _H), f32),
            pltpu.VMEM((_EK, _H), f32),
            pltpu.VMEM_SHARED((_NPAD, _H), f32),
            pltpu.VMEM_SHARED((_NPAD, _H), f32),
            pltpu.SemaphoreType.DMA,
        ],
        compiler_params=pltpu.CompilerParams(needs_layout_passes=False, use_tc_tiling_on_sc=False),
    )
    return run(hsplit_flat, src_pad, dst_pad, scores_pad)


def _fuse_body(self_w, fwd_w, bwd_w, comb_w, self_b, comb_b,
               ws_o, wd_o, wb_o, bias_o):
    c1 = comb_w[0, :_D, :]
    c2 = comb_w[0, _D:2 * _D, :]
    c3 = comb_w[0, 2 * _D:, :]
    f32 = jnp.float32
    ws_o[0] = jnp.dot(self_w[0], c1, preferred_element_type=f32)
    wf = jnp.dot(fwd_w[0], c2, preferred_element_type=f32)
    wb = jnp.dot(bwd_w[0], c3, preferred_element_type=f32)
    wd_o[0] = wf - wb
    wb_o[0] = wb
    bias_o[0, 0] = jnp.dot(self_b[0, 0], c1, preferred_element_type=f32) \
        + comb_b[0, 0]


def _fuse_weights(self_W, fwd_W, bwd_W, comb_W, self_b, comb_b):
    f32 = jnp.float32
    w_spec = pl.BlockSpec((1, _D, _D), lambda l: (l, 0, 0))
    b_spec = pl.BlockSpec((1, 1, _D), lambda l: (l, 0, 0))
    return pl.pallas_call(
        _fuse_body,
        grid=(_NL,),
        in_specs=[w_spec, w_spec, w_spec,
                  pl.BlockSpec((1, 3 * _D, _D), lambda l: (l, 0, 0)),
                  b_spec, b_spec],
        out_specs=[w_spec, w_spec, w_spec, b_spec],
        out_shape=[jax.ShapeDtypeStruct((_NL, _D, _D), f32),
                   jax.ShapeDtypeStruct((_NL, _D, _D), f32),
                   jax.ShapeDtypeStruct((_NL, _D, _D), f32),
                   jax.ShapeDtypeStruct((_NL, 1, _D), f32)],
    )(self_W, fwd_W, bwd_W, comb_W,
      self_b.reshape(_NL, 1, _D), comb_b.reshape(_NL, 1, _D))


def _prologue_body(x_ref, w_ref, b_ref, sw_ref, hs_o, sc_o):
    f32 = jnp.float32
    h = jnp.dot(x_ref[...], w_ref[...], preferred_element_type=f32)
    h = jax.nn.relu(h + b_ref[0])
    hs_o[0] = h[:, :_H]
    hs_o[1] = h[:, _H:]
    sc_o[0, 0] = jnp.dot(h, sw_ref[0, 0], preferred_element_type=f32)


def _prologue(x, emb_W, emb_b, sw0):
    f32 = jnp.float32
    return pl.pallas_call(
        _prologue_body,
        grid=(_NSTEPS,),
        in_specs=[pl.BlockSpec((_BN, _D), lambda i: (i, 0)),
                  pl.BlockSpec((_D, _D), lambda i: (0, 0)),
                  pl.BlockSpec((1, _D), lambda i: (0, 0)),
                  pl.BlockSpec((1, 1, _D), lambda i: (0, 0, 0))],
        out_specs=[pl.BlockSpec((2, _BN, _H), lambda i: (0, i, 0)),
                   pl.BlockSpec((1, 1, _BN), lambda i: (i, 0, 0))],
        out_shape=[jax.ShapeDtypeStruct((2, _N, _H), f32),
                   jax.ShapeDtypeStruct((_NSTEPS, 1, _BN), f32)],
    )(x, emb_W, emb_b.reshape(1, _D), sw0)


def _ln_relu_res(acc, h, g_ref, b_ref):
    mu = jnp.mean(acc, axis=-1, keepdims=True)
    var = jnp.mean((acc - mu) ** 2, axis=-1, keepdims=True)
    nrm = (acc - mu) / jnp.sqrt(var + 1e-5) * g_ref[0] + b_ref[0]
    return jax.nn.relu(nrm) + h


def _layer_body(hl, hr, pl_r, pr_r, ql, qr, ws, wd, wb, bias, g_ref, b_ref,
                sw_ref, hs_o, sc_o):
    f32 = jnp.float32
    h = jnp.concatenate([hl[0], hr[0]], axis=-1)
    p = jnp.concatenate([pl_r[0], pr_r[0]], axis=-1)
    q = jnp.concatenate([ql[0], qr[0]], axis=-1)
    acc = (jnp.dot(h, ws[...], preferred_element_type=f32)
           + jnp.dot(p, wd[...], preferred_element_type=f32)
           + jnp.dot(q, wb[...], preferred_element_type=f32)
           + bias[0])
    out = _ln_relu_res(acc, h, g_ref, b_ref)
    hs_o[0] = out[:, :_H]
    hs_o[1] = out[:, _H:]
    sc_o[0, 0] = jnp.dot(out, sw_ref[0, 0], preferred_element_type=f32)


def _layer(hs, p2, q2, ws, wd, wb, bias, ln_g1, ln_b1, sw_next):
    f32 = jnp.float32
    half = lambda j: pl.BlockSpec((1, _BN, _H), lambda i, j=j: (j, i, 0))
    wspec = pl.BlockSpec((_D, _D), lambda i: (0, 0))
    vspec = pl.BlockSpec((1, _D), lambda i: (0, 0))
    return pl.pallas_call(
        _layer_body,
        grid=(_NSTEPS,),
        in_specs=[half(0), half(1), half(0), half(1), half(0), half(1),
                  wspec, wspec, wspec, vspec, vspec, vspec,
                  pl.BlockSpec((1, 1, _D), lambda i: (0, 0, 0))],
        out_specs=[pl.BlockSpec((2, _BN, _H), lambda i: (0, i, 0)),
                   pl.BlockSpec((1, 1, _BN), lambda i: (i, 0, 0))],
        out_shape=[jax.ShapeDtypeStruct((2, _N, _H), f32),
                   jax.ShapeDtypeStruct((_NSTEPS, 1, _BN), f32)],
    )(hs, hs, p2, p2, q2, q2, ws, wd, wb, bias, ln_g1, ln_b1, sw_next)


def _final_body(hl, hr, pl_r, pr_r, ql, qr, ws, wd, wb, bias, g_ref, b_ref,
                batch_ref, pooled_o, sum_acc, cnt_acc):
    f32 = jnp.float32
    i = pl.program_id(0)
    h = jnp.concatenate([hl[0], hr[0]], axis=-1)
    p = jnp.concatenate([pl_r[0], pr_r[0]], axis=-1)
    q = jnp.concatenate([ql[0], qr[0]], axis=-1)
    acc = (jnp.dot(h, ws[...], preferred_element_type=f32)
           + jnp.dot(p, wd[...], preferred_element_type=f32)
           + jnp.dot(q, wb[...], preferred_element_type=f32)
           + bias[0])
    out = _ln_relu_res(acc, h, g_ref, b_ref)

    @pl.when(i == 0)
    def _():
        sum_acc[...] = jnp.zeros((_G, _D), f32)
        cnt_acc[...] = jnp.zeros((_G, 1), f32)

    gids = lax.broadcasted_iota(jnp.int32, (_G, _BN), 0)
    oh = (gids == batch_ref[0, 0][None, :]).astype(f32)
    sum_acc[...] += jnp.dot(oh, out, preferred_element_type=f32)
    cnt_acc[...] += jnp.sum(oh, axis=1, keepdims=True)

    @pl.when(i == _NSTEPS - 1)
    def _():
        pooled_o[...] = sum_acc[...] / jnp.maximum(cnt_acc[...], 1.0)


def _final_layer(hs, p2, q2, ws, wd, wb, bias, ln_g1, ln_b1, batch3d):
    f32 = jnp.float32
    half = lambda j: pl.BlockSpec((1, _BN, _H), lambda i, j=j: (j, i, 0))
    wspec = pl.BlockSpec((_D, _D), lambda i: (0, 0))
    vspec = pl.BlockSpec((1, _D), lambda i: (0, 0))
    return pl.pallas_call(
        _final_body,
        grid=(_NSTEPS,),
        in_specs=[half(0), half(1), half(0), half(1), half(0), half(1),
                  wspec, wspec, wspec, vspec, vspec, vspec,
                  pl.BlockSpec((1, 1, _BN), lambda i: (i, 0, 0))],
        out_specs=pl.BlockSpec((_G, _D), lambda i: (0, 0)),
        out_shape=jax.ShapeDtypeStruct((_G, _D), f32),
        scratch_shapes=[pltpu.VMEM((_G, _D), f32),
                        pltpu.VMEM((_G, 1), f32)],
    )(hs, hs, p2, p2, q2, q2, ws, wd, wb, bias, ln_g1, ln_b1, batch3d)


def kernel(x, edge_index, batch, emb_W, emb_b, score_W, score_b, fwd_W,
           bwd_W, self_W, self_b, comb_W, comb_b, ln_g, ln_b):
    src = edge_index[0].astype(jnp.int32)
    dst = edge_index[1].astype(jnp.int32)
    npad = _EPAD - _E
    src_pad = jnp.concatenate([src, jnp.zeros((npad,), jnp.int32)])
    dst_pad = jnp.concatenate([dst, jnp.full((npad,), _N, jnp.int32)])
    batch3d = batch.astype(jnp.int32).reshape(_NSTEPS, 1, _BN)
    sw = score_W.reshape(_NL, 1, 1, _D)

    ws_s, wd_s, wb_s, bias_s = _fuse_weights(self_W, fwd_W, bwd_W, comb_W,
                                             self_b, comb_b)

    hs, sc = _prologue(x, emb_W, emb_b, sw[0])
    for l in range(_NL):
        scores_pad = jnp.pad(sc.reshape(_N), (0, _NPAD - _N))
        p2, q2 = _sc_edge(hs.reshape(_NC * _N, _H), src_pad, dst_pad,
                          scores_pad)
        args = (hs, p2, q2, ws_s[l], wd_s[l], wb_s[l], bias_s[l],
                ln_g[l].reshape(1, _D), ln_b[l].reshape(1, _D))
        if l < _NL - 1:
            hs, sc = _layer(*args, sw[l + 1])
        else:
            pooled = _final_layer(*args, batch3d)
    return (pooled, 0)


# speedup vs baseline: 10.8181x; 1.1928x over previous
import jax
import jax.numpy as jnp
from jax import lax
from jax.experimental import pallas as pl
from jax.experimental.pallas import tpu as pltpu
from jax.experimental.pallas import tpu_sc as plsc

_N = 10000
_E = 320000
_D = 128
_H = 64
_NL = 3
_G = 64
_NC = 2
_NS = 16
_NPAD = 10240
_EPAD = 327680
_EPS = _EPAD // _NS
_EK = 128
_BN = 1000
_NSTEPS = _N // _BN


def _sc_edge_body(h_hbm, comb_hbm, scores_hbm, p_out, q_out,
                  scores_v, dst0, dst1, dst2,
                  comb0, comb1, comb2, srcg0, srcg1, srcg2,
                  rows0, rows1, rows2, wrows_v, p_acc, q_acc,
                  gsem0, gsem1, gsem2, csem0, csem1, csem2):
    cid = lax.axis_index("c")
    sid = lax.axis_index("s")
    nch = _EPS // _EK
    rows_bufs = (rows0, rows1, rows2)
    dst_bufs = (dst0, dst1, dst2)
    comb_bufs = (comb0, comb1, comb2)
    srcg_bufs = (srcg0, srcg1, srcg2)
    gsems = (gsem0, gsem1, gsem2)
    csems = (csem0, csem1, csem2)
    ndeep = len(rows_bufs)

    def _zrow(j, c):
        for t in range(_H // 16):
            wrows_v[j, pl.ds(t * 16, 16)] = jnp.zeros((16,), jnp.float32)
        return c
    lax.fori_loop(0, _EK, _zrow, 0)

    rows_per_tile = _NPAD // _NS
    def _zacc(ci, c):
        base = sid * rows_per_tile + ci * _EK
        pltpu.sync_copy(wrows_v, p_acc.at[pl.ds(base, _EK)])
        pltpu.sync_copy(wrows_v, q_acc.at[pl.ds(base, _EK)])
        return c
    lax.fori_loop(0, rows_per_tile // _EK, _zacc, 0)

    pltpu.sync_copy(scores_hbm, scores_v)

    cbase = sid * nch
    toff = cid * _N

    def _fill_srcg(comb_b, srcg_b, dst_b):
        def _fg(g, cc):
            sl = pl.ds(g * 16, 16)
            v = comb_b[sl]
            srcg_b[sl] = (v & 16383) + toff
            dst_b[sl] = lax.shift_right_logical(v, 14)
            return cc
        lax.fori_loop(0, _EK // 16, _fg, 0)

    plsc.subcore_barrier()

    for q in range(ndeep):
        pltpu.sync_copy(comb_hbm.at[cbase + q], comb_bufs[q])
        _fill_srcg(comb_bufs[q], srcg_bufs[q], dst_bufs[q])
        pltpu.async_copy(h_hbm.at[srcg_bufs[q]], rows_bufs[q], gsems[q])
        pltpu.async_copy(comb_hbm.at[cbase + q + ndeep], comb_bufs[q],
                         csems[q])

    def _process(c, rows_b, comb_b, srcg_b, dst_b, gsem_b, csem_b):
        pltpu.make_async_copy(h_hbm.at[pl.ds(0, _EK)], rows_b, gsem_b).wait()

        def _grp(g, cc):
            sl = pl.ds(g * 16, 16)
            si = srcg_b[sl] - toff
            di = dst_b[sl]
            ss = plsc.load_gather(scores_v, [si])
            sd = plsc.load_gather(scores_v, [di])
            al = 1.0 / (1.0 + jnp.exp(sd - ss))
            for e in range(16):
                a = al[e]
                j = g * 16 + e
                for t in range(_H // 16):
                    sl2 = pl.ds(t * 16, 16)
                    wrows_v[j, sl2] = rows_b[j, sl2] * a
            return cc
        lax.fori_loop(0, _EK // 16, _grp, 0)

        pltpu.sync_copy(wrows_v, p_acc.at[dst_b], add=True)
        pltpu.sync_copy(rows_b, q_acc.at[dst_b], add=True)

        @pl.when(c + ndeep < nch)
        def _():
            pltpu.make_async_copy(comb_hbm.at[cbase], comb_b, csem_b).wait()
            _fill_srcg(comb_b, srcg_b, dst_b)
            pltpu.async_copy(h_hbm.at[srcg_b], rows_b, gsem_b)

            @pl.when(c + 2 * ndeep < nch)
            def _():
                pltpu.async_copy(comb_hbm.at[cbase + c + 2 * ndeep], comb_b,
                                 csem_b)

    def _quad(i4, c):
        for q in range(ndeep):
            _process(i4 * ndeep + q, rows_bufs[q], comb_bufs[q],
                     srcg_bufs[q], dst_bufs[q], gsems[q], csems[q])
        return c
    lax.fori_loop(0, nch // ndeep, _quad, 0)
    for r in range(nch - (nch // ndeep) * ndeep):
        c = (nch // ndeep) * ndeep + r
        q = c % ndeep
        _process(c, rows_bufs[q], comb_bufs[q], srcg_bufs[q], dst_bufs[q],
                 gsems[q], csems[q])

    plsc.subcore_barrier()

    obase = sid * rows_per_tile
    nvalid = _N - (_NS - 1) * rows_per_tile

    @pl.when(sid < _NS - 1)
    def _():
        pltpu.sync_copy(p_acc.at[pl.ds(obase, rows_per_tile)],
                        p_out.at[cid, pl.ds(obase, rows_per_tile)])
        pltpu.sync_copy(q_acc.at[pl.ds(obase, rows_per_tile)],
                        q_out.at[cid, pl.ds(obase, rows_per_tile)])

    @pl.when(sid == _NS - 1)
    def _():
        pltpu.sync_copy(p_acc.at[pl.ds(obase, nvalid)],
                        p_out.at[cid, pl.ds(obase, nvalid)])
        pltpu.sync_copy(q_acc.at[pl.ds(obase, nvalid)],
                        q_out.at[cid, pl.ds(obase, nvalid)])


def _sc_edge(hsplit_flat, comb2d, scores_pad):
    mesh = plsc.VectorSubcoreMesh(core_axis_name="c", subcore_axis_name="s")
    f32 = jnp.float32
    i32 = jnp.int32
    run = pl.kernel(
        _sc_edge_body,
        out_type=(jax.ShapeDtypeStruct((_NC, _N, _H), f32),
                  jax.ShapeDtypeStruct((_NC, _N, _H), f32)),
        mesh=mesh,
        scratch_types=[
            pltpu.VMEM((_NPAD,), f32),
            pltpu.VMEM((_EK,), i32),
            pltpu.VMEM((_EK,), i32),
            pltpu.VMEM((_EK,), i32),
            pltpu.VMEM((_EK,), i32),
            pltpu.VMEM((_EK,), i32),
            pltpu.VMEM((_EK,), i32),
            pltpu.VMEM((_EK,), i32),
            pltpu.VMEM((_EK,), i32),
            pltpu.VMEM((_EK,), i32),
            pltpu.VMEM((_EK, _H), f32),
            pltpu.VMEM((_EK, _H), f32),
            pltpu.VMEM((_EK, _H), f32),
            pltpu.VMEM((_EK, _H), f32),
            pltpu.VMEM_SHARED((_NPAD, _H), f32),
            pltpu.VMEM_SHARED((_NPAD, _H), f32),
            pltpu.SemaphoreType.DMA,
            pltpu.SemaphoreType.DMA,
            pltpu.SemaphoreType.DMA,
            pltpu.SemaphoreType.DMA,
            pltpu.SemaphoreType.DMA,
            pltpu.SemaphoreType.DMA,
        ],
        compiler_params=pltpu.CompilerParams(needs_layout_passes=False, use_tc_tiling_on_sc=False),
    )
    return run(hsplit_flat, comb2d, scores_pad)


def _fuse_body(self_w, fwd_w, bwd_w, comb_w, self_b, comb_b,
               ws_o, wd_o, wb_o, bias_o):
    c1 = comb_w[0, :_D, :]
    c2 = comb_w[0, _D:2 * _D, :]
    c3 = comb_w[0, 2 * _D:, :]
    f32 = jnp.float32
    ws_o[0] = jnp.dot(self_w[0], c1, preferred_element_type=f32)
    wf = jnp.dot(fwd_w[0], c2, preferred_element_type=f32)
    wb = jnp.dot(bwd_w[0], c3, preferred_element_type=f32)
    wd_o[0] = wf - wb
    wb_o[0] = wb
    bias_o[0, 0] = jnp.dot(self_b[0, 0], c1, preferred_element_type=f32) \
        + comb_b[0, 0]


def _fuse_weights(self_W, fwd_W, bwd_W, comb_W, self_b, comb_b):
    f32 = jnp.float32
    w_spec = pl.BlockSpec((1, _D, _D), lambda l: (l, 0, 0))
    b_spec = pl.BlockSpec((1, 1, _D), lambda l: (l, 0, 0))
    return pl.pallas_call(
        _fuse_body,
        grid=(_NL,),
        in_specs=[w_spec, w_spec, w_spec,
                  pl.BlockSpec((1, 3 * _D, _D), lambda l: (l, 0, 0)),
                  b_spec, b_spec],
        out_specs=[w_spec, w_spec, w_spec, b_spec],
        out_shape=[jax.ShapeDtypeStruct((_NL, _D, _D), f32),
                   jax.ShapeDtypeStruct((_NL, _D, _D), f32),
                   jax.ShapeDtypeStruct((_NL, _D, _D), f32),
                   jax.ShapeDtypeStruct((_NL, 1, _D), f32)],
    )(self_W, fwd_W, bwd_W, comb_W,
      self_b.reshape(_NL, 1, _D), comb_b.reshape(_NL, 1, _D))


def _prologue_body(x_ref, w_ref, b_ref, sw_ref, hs_o, sc_o):
    f32 = jnp.float32
    h = jnp.dot(x_ref[...], w_ref[...], preferred_element_type=f32)
    h = jax.nn.relu(h + b_ref[0])
    hs_o[0] = h[:, :_H]
    hs_o[1] = h[:, _H:]
    sc_o[0, 0] = jnp.dot(h, sw_ref[0, 0], preferred_element_type=f32)


def _prologue(x, emb_W, emb_b, sw0):
    f32 = jnp.float32
    return pl.pallas_call(
        _prologue_body,
        grid=(_NSTEPS,),
        in_specs=[pl.BlockSpec((_BN, _D), lambda i: (i, 0)),
                  pl.BlockSpec((_D, _D), lambda i: (0, 0)),
                  pl.BlockSpec((1, _D), lambda i: (0, 0)),
                  pl.BlockSpec((1, 1, _D), lambda i: (0, 0, 0))],
        out_specs=[pl.BlockSpec((2, _BN, _H), lambda i: (0, i, 0)),
                   pl.BlockSpec((1, 1, _BN), lambda i: (i, 0, 0))],
        out_shape=[jax.ShapeDtypeStruct((2, _N, _H), f32),
                   jax.ShapeDtypeStruct((_NSTEPS, 1, _BN), f32)],
    )(x, emb_W, emb_b.reshape(1, _D), sw0)


def _ln_relu_res(acc, h, g_ref, b_ref):
    mu = jnp.mean(acc, axis=-1, keepdims=True)
    var = jnp.mean((acc - mu) ** 2, axis=-1, keepdims=True)
    nrm = (acc - mu) / jnp.sqrt(var + 1e-5) * g_ref[0] + b_ref[0]
    return jax.nn.relu(nrm) + h


def _layer_body(hl, hr, pl_r, pr_r, ql, qr, ws, wd, wb, bias, g_ref, b_ref,
                sw_ref, hs_o, sc_o):
    f32 = jnp.float32
    h = jnp.concatenate([hl[0], hr[0]], axis=-1)
    p = jnp.concatenate([pl_r[0], pr_r[0]], axis=-1)
    q = jnp.concatenate([ql[0], qr[0]], axis=-1)
    acc = (jnp.dot(h, ws[...], preferred_element_type=f32)
           + jnp.dot(p, wd[...], preferred_element_type=f32)
           + jnp.dot(q, wb[...], preferred_element_type=f32)
           + bias[0])
    out = _ln_relu_res(acc, h, g_ref, b_ref)
    hs_o[0] = out[:, :_H]
    hs_o[1] = out[:, _H:]
    sc_o[0, 0] = jnp.dot(out, sw_ref[0, 0], preferred_element_type=f32)


def _layer(hs, p2, q2, ws, wd, wb, bias, ln_g1, ln_b1, sw_next):
    f32 = jnp.float32
    half = lambda j: pl.BlockSpec((1, _BN, _H), lambda i, j=j: (j, i, 0))
    wspec = pl.BlockSpec((_D, _D), lambda i: (0, 0))
    vspec = pl.BlockSpec((1, _D), lambda i: (0, 0))
    return pl.pallas_call(
        _layer_body,
        grid=(_NSTEPS,),
        in_specs=[half(0), half(1), half(0), half(1), half(0), half(1),
                  wspec, wspec, wspec, vspec, vspec, vspec,
                  pl.BlockSpec((1, 1, _D), lambda i: (0, 0, 0))],
        out_specs=[pl.BlockSpec((2, _BN, _H), lambda i: (0, i, 0)),
                   pl.BlockSpec((1, 1, _BN), lambda i: (i, 0, 0))],
        out_shape=[jax.ShapeDtypeStruct((2, _N, _H), f32),
                   jax.ShapeDtypeStruct((_NSTEPS, 1, _BN), f32)],
    )(hs, hs, p2, p2, q2, q2, ws, wd, wb, bias, ln_g1, ln_b1, sw_next)


def _final_body(hl, hr, pl_r, pr_r, ql, qr, ws, wd, wb, bias, g_ref, b_ref,
                batch_ref, pooled_o, sum_acc, cnt_acc):
    f32 = jnp.float32
    i = pl.program_id(0)
    h = jnp.concatenate([hl[0], hr[0]], axis=-1)
    p = jnp.concatenate([pl_r[0], pr_r[0]], axis=-1)
    q = jnp.concatenate([ql[0], qr[0]], axis=-1)
    acc = (jnp.dot(h, ws[...], preferred_element_type=f32)
           + jnp.dot(p, wd[...], preferred_element_type=f32)
           + jnp.dot(q, wb[...], preferred_element_type=f32)
           + bias[0])
    out = _ln_relu_res(acc, h, g_ref, b_ref)

    @pl.when(i == 0)
    def _():
        sum_acc[...] = jnp.zeros((_G, _D), f32)
        cnt_acc[...] = jnp.zeros((_G, 1), f32)

    gids = lax.broadcasted_iota(jnp.int32, (_G, _BN), 0)
    oh = (gids == batch_ref[0, 0][None, :]).astype(f32)
    sum_acc[...] += jnp.dot(oh, out, preferred_element_type=f32)
    cnt_acc[...] += jnp.sum(oh, axis=1, keepdims=True)

    @pl.when(i == _NSTEPS - 1)
    def _():
        pooled_o[...] = sum_acc[...] / jnp.maximum(cnt_acc[...], 1.0)


def _final_layer(hs, p2, q2, ws, wd, wb, bias, ln_g1, ln_b1, batch3d):
    f32 = jnp.float32
    half = lambda j: pl.BlockSpec((1, _BN, _H), lambda i, j=j: (j, i, 0))
    wspec = pl.BlockSpec((_D, _D), lambda i: (0, 0))
    vspec = pl.BlockSpec((1, _D), lambda i: (0, 0))
    return pl.pallas_call(
        _final_body,
        grid=(_NSTEPS,),
        in_specs=[half(0), half(1), half(0), half(1), half(0), half(1),
                  wspec, wspec, wspec, vspec, vspec, vspec,
                  pl.BlockSpec((1, 1, _BN), lambda i: (i, 0, 0))],
        out_specs=pl.BlockSpec((_G, _D), lambda i: (0, 0)),
        out_shape=jax.ShapeDtypeStruct((_G, _D), f32),
        scratch_shapes=[pltpu.VMEM((_G, _D), f32),
                        pltpu.VMEM((_G, 1), f32)],
    )(hs, hs, p2, p2, q2, q2, ws, wd, wb, bias, ln_g1, ln_b1, batch3d)


def kernel(x, edge_index, batch, emb_W, emb_b, score_W, score_b, fwd_W,
           bwd_W, self_W, self_b, comb_W, comb_b, ln_g, ln_b):
    src = edge_index[0].astype(jnp.int32)
    dst = edge_index[1].astype(jnp.int32)
    npad = _EPAD - _E
    comb = src + dst * 16384
    comb2d = jnp.concatenate(
        [comb, jnp.full((npad,), _N * 16384, jnp.int32)]).reshape(-1, _EK)
    batch3d = batch.astype(jnp.int32).reshape(_NSTEPS, 1, _BN)
    sw = score_W.reshape(_NL, 1, 1, _D)

    ws_s, wd_s, wb_s, bias_s = _fuse_weights(self_W, fwd_W, bwd_W, comb_W,
                                             self_b, comb_b)

    hs, sc = _prologue(x, emb_W, emb_b, sw[0])
    for l in range(_NL):
        scores_pad = jnp.pad(sc.reshape(_N), (0, _NPAD - _N))
        p2, q2 = _sc_edge(hs.reshape(_NC * _N, _H), comb2d, scores_pad)
        args = (hs, p2, q2, ws_s[l], wd_s[l], wb_s[l], bias_s[l],
                ln_g[l].reshape(1, _D), ln_b[l].reshape(1, _D))
        if l < _NL - 1:
            hs, sc = _layer(*args, sw[l + 1])
        else:
            pooled = _final_layer(*args, batch3d)
    return (pooled, 0)


# speedup vs baseline: 11.2200x; 1.0372x over previous
import jax
import jax.numpy as jnp
from jax import lax
from jax.experimental import pallas as pl
from jax.experimental.pallas import tpu as pltpu
from jax.experimental.pallas import tpu_sc as plsc

_N = 10000
_E = 320000
_D = 128
_H = 64
_NL = 3
_G = 64
_NC = 2
_NS = 16
_NPAD = 10240
_EPAD = 327680
_EPS = _EPAD // _NS
_EK = 128
_BN = 1000
_NSTEPS = _N // _BN


def _sc_edge_body(h_hbm, comb_hbm, scores2_hbm, p_out, q_out,
                  comb0, comb1, comb2, comb3,
                  srcg0, srcg1, srcg2, srcg3,
                  dst0, dst1, dst2, dst3,
                  ssv0, ssv1, ssv2, ssv3,
                  sdv0, sdv1, sdv2, sdv3,
                  rows0, rows1, rows2, rows3, wrows_v, p_acc, q_acc,
                  gsem0, gsem1, gsem2, gsem3,
                  csem0, csem1, csem2, csem3):
    cid = lax.axis_index("c")
    sid = lax.axis_index("s")
    nch = _EPS // _EK
    comb_bufs = (comb0, comb1, comb2, comb3)
    srcg_bufs = (srcg0, srcg1, srcg2, srcg3)
    dst_bufs = (dst0, dst1, dst2, dst3)
    ssv_bufs = (ssv0, ssv1, ssv2, ssv3)
    sdv_bufs = (sdv0, sdv1, sdv2, sdv3)
    rows_bufs = (rows0, rows1, rows2, rows3)
    gsems = (gsem0, gsem1, gsem2, gsem3)
    csems = (csem0, csem1, csem2, csem3)
    ndeep = len(rows_bufs)

    def _zrow(j, c):
        for t in range(_H // 16):
            wrows_v[j, pl.ds(t * 16, 16)] = jnp.zeros((16,), jnp.float32)
        return c
    lax.fori_loop(0, _EK, _zrow, 0)

    rows_per_tile = _NPAD // _NS
    def _zacc(ci, c):
        base = sid * rows_per_tile + ci * _EK
        pltpu.sync_copy(wrows_v, p_acc.at[pl.ds(base, _EK)])
        pltpu.sync_copy(wrows_v, q_acc.at[pl.ds(base, _EK)])
        return c
    lax.fori_loop(0, rows_per_tile // _EK, _zacc, 0)

    cbase = sid * nch
    toff = cid * _N

    def _fill_srcg(comb_b, srcg_b, dst_b):
        def _fg(g, cc):
            sl = pl.ds(g * 16, 16)
            v = comb_b[sl]
            srcg_b[sl] = (v & 16383) + toff
            dst_b[sl] = lax.shift_right_logical(v, 14)
            return cc
        lax.fori_loop(0, _EK // 16, _fg, 0)

    def _fire(q, srcg_b, dst_b, gsem_b):
        pltpu.async_copy(h_hbm.at[srcg_b], rows_bufs[q], gsem_b)
        pltpu.async_copy(scores2_hbm.at[srcg_b], ssv_bufs[q], gsem_b)
        pltpu.async_copy(scores2_hbm.at[dst_b], sdv_bufs[q], gsem_b)

    plsc.subcore_barrier()

    for q in range(ndeep):
        pltpu.sync_copy(comb_hbm.at[cbase + q], comb_bufs[q])
        _fill_srcg(comb_bufs[q], srcg_bufs[q], dst_bufs[q])
        _fire(q, srcg_bufs[q], dst_bufs[q], gsems[q])
        pltpu.async_copy(comb_hbm.at[cbase + q + ndeep], comb_bufs[q],
                         csems[q])

    def _process(c, q):
        rows_b = rows_bufs[q]
        comb_b = comb_bufs[q]
        srcg_b = srcg_bufs[q]
        dst_b = dst_bufs[q]
        gsem_b = gsems[q]
        csem_b = csems[q]
        pltpu.make_async_copy(h_hbm.at[pl.ds(0, _EK)], rows_b,
                              gsem_b).wait()
        pltpu.make_async_copy(scores2_hbm.at[pl.ds(0, _EK)], ssv_bufs[q],
                              gsem_b).wait()
        pltpu.make_async_copy(scores2_hbm.at[pl.ds(0, _EK)], sdv_bufs[q],
                              gsem_b).wait()

        def _grp(g, cc):
            sl = pl.ds(g * 16, 16)
            ss = ssv_bufs[q][sl]
            sd = sdv_bufs[q][sl]
            al = 1.0 / (1.0 + jnp.exp(sd - ss))
            for e in range(16):
                a = al[e]
                j = g * 16 + e
                for t in range(_H // 16):
                    sl2 = pl.ds(t * 16, 16)
                    wrows_v[j, sl2] = rows_b[j, sl2] * a
            return cc
        lax.fori_loop(0, _EK // 16, _grp, 0)

        pltpu.sync_copy(wrows_v, p_acc.at[dst_b], add=True)
        pltpu.sync_copy(rows_b, q_acc.at[dst_b], add=True)

        @pl.when(c + ndeep < nch)
        def _():
            pltpu.make_async_copy(comb_hbm.at[cbase], comb_b, csem_b).wait()
            _fill_srcg(comb_b, srcg_b, dst_b)
            _fire(q, srcg_b, dst_b, gsem_b)

            @pl.when(c + 2 * ndeep < nch)
            def _():
                pltpu.async_copy(comb_hbm.at[cbase + c + 2 * ndeep], comb_b,
                                 csem_b)

    def _quad(i4, c):
        for q in range(ndeep):
            _process(i4 * ndeep + q, q)
        return c
    lax.fori_loop(0, nch // ndeep, _quad, 0)

    plsc.subcore_barrier()

    obase = sid * rows_per_tile
    nvalid = _N - (_NS - 1) * rows_per_tile

    @pl.when(sid < _NS - 1)
    def _():
        pltpu.sync_copy(p_acc.at[pl.ds(obase, rows_per_tile)],
                        p_out.at[cid, pl.ds(obase, rows_per_tile)])
        pltpu.sync_copy(q_acc.at[pl.ds(obase, rows_per_tile)],
                        q_out.at[cid, pl.ds(obase, rows_per_tile)])

    @pl.when(sid == _NS - 1)
    def _():
        pltpu.sync_copy(p_acc.at[pl.ds(obase, nvalid)],
                        p_out.at[cid, pl.ds(obase, nvalid)])
        pltpu.sync_copy(q_acc.at[pl.ds(obase, nvalid)],
                        q_out.at[cid, pl.ds(obase, nvalid)])


def _sc_edge(hsplit_flat, comb2d, scores2):
    mesh = plsc.VectorSubcoreMesh(core_axis_name="c", subcore_axis_name="s")
    f32 = jnp.float32
    i32 = jnp.int32
    ndeep = 4
    scratch = []
    for _ in range(ndeep):
        scratch.append(pltpu.VMEM((_EK,), i32))
    for _ in range(ndeep):
        scratch.append(pltpu.VMEM((_EK,), i32))
    for _ in range(ndeep):
        scratch.append(pltpu.VMEM((_EK,), i32))
    for _ in range(ndeep):
        scratch.append(pltpu.VMEM((_EK,), f32))
    for _ in range(ndeep):
        scratch.append(pltpu.VMEM((_EK,), f32))
    for _ in range(ndeep):
        scratch.append(pltpu.VMEM((_EK, _H), f32))
    scratch.append(pltpu.VMEM((_EK, _H), f32))
    scratch.append(pltpu.VMEM_SHARED((_NPAD, _H), f32))
    scratch.append(pltpu.VMEM_SHARED((_NPAD, _H), f32))
    for _ in range(2 * ndeep):
        scratch.append(pltpu.SemaphoreType.DMA)
    run = pl.kernel(
        _sc_edge_body,
        out_type=(jax.ShapeDtypeStruct((_NC, _N, _H), f32),
                  jax.ShapeDtypeStruct((_NC, _N, _H), f32)),
        mesh=mesh,
        scratch_types=scratch,
        compiler_params=pltpu.CompilerParams(needs_layout_passes=False,
                                             use_tc_tiling_on_sc=False),
    )
    return run(hsplit_flat, comb2d, scores2)


def _fuse_body(self_w, fwd_w, bwd_w, comb_w, self_b, comb_b,
               ws_o, wd_o, wb_o, bias_o):
    c1 = comb_w[0, :_D, :]
    c2 = comb_w[0, _D:2 * _D, :]
    c3 = comb_w[0, 2 * _D:, :]
    f32 = jnp.float32
    ws_o[0] = jnp.dot(self_w[0], c1, preferred_element_type=f32)
    wf = jnp.dot(fwd_w[0], c2, preferred_element_type=f32)
    wb = jnp.dot(bwd_w[0], c3, preferred_element_type=f32)
    wd_o[0] = wf - wb
    wb_o[0] = wb
    bias_o[0, 0] = jnp.dot(self_b[0, 0], c1, preferred_element_type=f32) \
        + comb_b[0, 0]


def _fuse_weights(self_W, fwd_W, bwd_W, comb_W, self_b, comb_b):
    f32 = jnp.float32
    w_spec = pl.BlockSpec((1, _D, _D), lambda l: (l, 0, 0))
    b_spec = pl.BlockSpec((1, 1, _D), lambda l: (l, 0, 0))
    return pl.pallas_call(
        _fuse_body,
        grid=(_NL,),
        in_specs=[w_spec, w_spec, w_spec,
                  pl.BlockSpec((1, 3 * _D, _D), lambda l: (l, 0, 0)),
                  b_spec, b_spec],
        out_specs=[w_spec, w_spec, w_spec, b_spec],
        out_shape=[jax.ShapeDtypeStruct((_NL, _D, _D), f32),
                   jax.ShapeDtypeStruct((_NL, _D, _D), f32),
                   jax.ShapeDtypeStruct((_NL, _D, _D), f32),
                   jax.ShapeDtypeStruct((_NL, 1, _D), f32)],
    )(self_W, fwd_W, bwd_W, comb_W,
      self_b.reshape(_NL, 1, _D), comb_b.reshape(_NL, 1, _D))


def _prologue_body(x_ref, w_ref, b_ref, sw_ref, hs_o, sc_o):
    f32 = jnp.float32
    h = jnp.dot(x_ref[...], w_ref[...], preferred_element_type=f32)
    h = jax.nn.relu(h + b_ref[0])
    hs_o[0] = h[:, :_H]
    hs_o[1] = h[:, _H:]
    sc_o[0, 0] = jnp.dot(h, sw_ref[0, 0], preferred_element_type=f32)


def _prologue(x, emb_W, emb_b, sw0):
    f32 = jnp.float32
    return pl.pallas_call(
        _prologue_body,
        grid=(_NSTEPS,),
        in_specs=[pl.BlockSpec((_BN, _D), lambda i: (i, 0)),
                  pl.BlockSpec((_D, _D), lambda i: (0, 0)),
                  pl.BlockSpec((1, _D), lambda i: (0, 0)),
                  pl.BlockSpec((1, 1, _D), lambda i: (0, 0, 0))],
        out_specs=[pl.BlockSpec((2, _BN, _H), lambda i: (0, i, 0)),
                   pl.BlockSpec((1, 1, _BN), lambda i: (i, 0, 0))],
        out_shape=[jax.ShapeDtypeStruct((2, _N, _H), f32),
                   jax.ShapeDtypeStruct((_NSTEPS, 1, _BN), f32)],
    )(x, emb_W, emb_b.reshape(1, _D), sw0)


def _ln_relu_res(acc, h, g_ref, b_ref):
    mu = jnp.mean(acc, axis=-1, keepdims=True)
    var = jnp.mean((acc - mu) ** 2, axis=-1, keepdims=True)
    nrm = (acc - mu) / jnp.sqrt(var + 1e-5) * g_ref[0] + b_ref[0]
    return jax.nn.relu(nrm) + h


def _layer_body(hl, hr, pl_r, pr_r, ql, qr, ws, wd, wb, bias, g_ref, b_ref,
                sw_ref, hs_o, sc_o):
    f32 = jnp.float32
    h = jnp.concatenate([hl[0], hr[0]], axis=-1)
    p = jnp.concatenate([pl_r[0], pr_r[0]], axis=-1)
    q = jnp.concatenate([ql[0], qr[0]], axis=-1)
    acc = (jnp.dot(h, ws[...], preferred_element_type=f32)
           + jnp.dot(p, wd[...], preferred_element_type=f32)
           + jnp.dot(q, wb[...], preferred_element_type=f32)
           + bias[0])
    out = _ln_relu_res(acc, h, g_ref, b_ref)
    hs_o[0] = out[:, :_H]
    hs_o[1] = out[:, _H:]
    sc_o[0, 0] = jnp.dot(out, sw_ref[0, 0], preferred_element_type=f32)


def _layer(hs, p2, q2, ws, wd, wb, bias, ln_g1, ln_b1, sw_next):
    f32 = jnp.float32
    half = lambda j: pl.BlockSpec((1, _BN, _H), lambda i, j=j: (j, i, 0))
    wspec = pl.BlockSpec((_D, _D), lambda i: (0, 0))
    vspec = pl.BlockSpec((1, _D), lambda i: (0, 0))
    return pl.pallas_call(
        _layer_body,
        grid=(_NSTEPS,),
        in_specs=[half(0), half(1), half(0), half(1), half(0), half(1),
                  wspec, wspec, wspec, vspec, vspec, vspec,
                  pl.BlockSpec((1, 1, _D), lambda i: (0, 0, 0))],
        out_specs=[pl.BlockSpec((2, _BN, _H), lambda i: (0, i, 0)),
                   pl.BlockSpec((1, 1, _BN), lambda i: (i, 0, 0))],
        out_shape=[jax.ShapeDtypeStruct((2, _N, _H), f32),
                   jax.ShapeDtypeStruct((_NSTEPS, 1, _BN), f32)],
    )(hs, hs, p2, p2, q2, q2, ws, wd, wb, bias, ln_g1, ln_b1, sw_next)


def _final_body(hl, hr, pl_r, pr_r, ql, qr, ws, wd, wb, bias, g_ref, b_ref,
                batch_ref, pooled_o, sum_acc, cnt_acc):
    f32 = jnp.float32
    i = pl.program_id(0)
    h = jnp.concatenate([hl[0], hr[0]], axis=-1)
    p = jnp.concatenate([pl_r[0], pr_r[0]], axis=-1)
    q = jnp.concatenate([ql[0], qr[0]], axis=-1)
    acc = (jnp.dot(h, ws[...], preferred_element_type=f32)
           + jnp.dot(p, wd[...], preferred_element_type=f32)
           + jnp.dot(q, wb[...], preferred_element_type=f32)
           + bias[0])
    out = _ln_relu_res(acc, h, g_ref, b_ref)

    @pl.when(i == 0)
    def _():
        sum_acc[...] = jnp.zeros((_G, _D), f32)
        cnt_acc[...] = jnp.zeros((_G, 1), f32)

    gids = lax.broadcasted_iota(jnp.int32, (_G, _BN), 0)
    oh = (gids == batch_ref[0, 0][None, :]).astype(f32)
    sum_acc[...] += jnp.dot(oh, out, preferred_element_type=f32)
    cnt_acc[...] += jnp.sum(oh, axis=1, keepdims=True)

    @pl.when(i == _NSTEPS - 1)
    def _():
        pooled_o[...] = sum_acc[...] / jnp.maximum(cnt_acc[...], 1.0)


def _final_layer(hs, p2, q2, ws, wd, wb, bias, ln_g1, ln_b1, batch3d):
    f32 = jnp.float32
    half = lambda j: pl.BlockSpec((1, _BN, _H), lambda i, j=j: (j, i, 0))
    wspec = pl.BlockSpec((_D, _D), lambda i: (0, 0))
    vspec = pl.BlockSpec((1, _D), lambda i: (0, 0))
    return pl.pallas_call(
        _final_body,
        grid=(_NSTEPS,),
        in_specs=[half(0), half(1), half(0), half(1), half(0), half(1),
                  wspec, wspec, wspec, vspec, vspec, vspec,
                  pl.BlockSpec((1, 1, _BN), lambda i: (i, 0, 0))],
        out_specs=pl.BlockSpec((_G, _D), lambda i: (0, 0)),
        out_shape=jax.ShapeDtypeStruct((_G, _D), f32),
        scratch_shapes=[pltpu.VMEM((_G, _D), f32),
                        pltpu.VMEM((_G, 1), f32)],
    )(hs, hs, p2, p2, q2, q2, ws, wd, wb, bias, ln_g1, ln_b1, batch3d)


def kernel(x, edge_index, batch, emb_W, emb_b, score_W, score_b, fwd_W,
           bwd_W, self_W, self_b, comb_W, comb_b, ln_g, ln_b):
    src = edge_index[0].astype(jnp.int32)
    dst = edge_index[1].astype(jnp.int32)
    npad = _EPAD - _E
    comb = src + dst * 16384
    comb2d = jnp.concatenate(
        [comb, jnp.full((npad,), _N * 16384, jnp.int32)]).reshape(-1, _EK)
    batch3d = batch.astype(jnp.int32).reshape(_NSTEPS, 1, _BN)
    sw = score_W.reshape(_NL, 1, 1, _D)

    ws_s, wd_s, wb_s, bias_s = _fuse_weights(self_W, fwd_W, bwd_W, comb_W,
                                             self_b, comb_b)

    hs, sc = _prologue(x, emb_W, emb_b, sw[0])
    for l in range(_NL):
        sflat = sc.reshape(_N)
        scores2 = jnp.concatenate(
            [sflat, sflat, jnp.zeros((2 * _NPAD - 2 * _N,), jnp.float32)])
        p2, q2 = _sc_edge(hs.reshape(_NC * _N, _H), comb2d, scores2)
        args = (hs, p2, q2, ws_s[l], wd_s[l], wb_s[l], bias_s[l],
                ln_g[l].reshape(1, _D), ln_b[l].reshape(1, _D))
        if l < _NL - 1:
            hs, sc = _layer(*args, sw[l + 1])
        else:
            pooled = _final_layer(*args, batch3d)
    return (pooled, 0)


# speedup vs baseline: 12.7103x; 1.1328x over previous
import jax
import jax.numpy as jnp
from jax import lax
from jax.experimental import pallas as pl
from jax.experimental.pallas import tpu as pltpu
from jax.experimental.pallas import tpu_sc as plsc

_N = 10000
_E = 320000
_D = 128
_H = 64
_NL = 3
_G = 64
_NC = 2
_NS = 16
_NPAD = 10240
_EPAD = 327680
_EPS = _EPAD // _NS
_EK = 128
_BN = 1000
_NSTEPS = _N // _BN


def _sc_edge_body(h_hbm, comb_hbm, scores2_hbm, p_out, q_out,
                  comb0, comb1, comb2,
                  srcg0, srcg1, srcg2,
                  dst0, dst1, dst2,
                  ssv0, ssv1, ssv2,
                  sdv0, sdv1, sdv2,
                  rows0, rows1, rows2, wra, wrb, pdst0, pdst1,
                  p_acc, q_acc,
                  gsem0, gsem1, gsem2,
                  csem0, csem1, csem2,
                  qsem0, qsem1, qsem2,
                  psem0, psem1):
    cid = lax.axis_index("c")
    sid = lax.axis_index("s")
    nch = _EPS // _EK
    comb_bufs = (comb0, comb1, comb2)
    srcg_bufs = (srcg0, srcg1, srcg2)
    dst_bufs = (dst0, dst1, dst2)
    ssv_bufs = (ssv0, ssv1, ssv2)
    sdv_bufs = (sdv0, sdv1, sdv2)
    rows_bufs = (rows0, rows1, rows2)
    wr_bufs = (wra, wrb)
    pdst_bufs = (pdst0, pdst1)
    gsems = (gsem0, gsem1, gsem2)
    csems = (csem0, csem1, csem2)
    qsems = (qsem0, qsem1, qsem2)
    psems = (psem0, psem1)
    ndeep = 3

    def _zrow(j, c):
        for t in range(_H // 16):
            wra[j, pl.ds(t * 16, 16)] = jnp.zeros((16,), jnp.float32)
        return c
    lax.fori_loop(0, _EK, _zrow, 0)

    rows_per_tile = _NPAD // _NS
    def _zacc(ci, c):
        base = sid * rows_per_tile + ci * _EK
        pltpu.sync_copy(wra, p_acc.at[pl.ds(base, _EK)])
        pltpu.sync_copy(wra, q_acc.at[pl.ds(base, _EK)])
        return c
    lax.fori_loop(0, rows_per_tile // _EK, _zacc, 0)

    cbase = sid * nch
    toff = cid * _N

    def _fill_srcg(comb_b, srcg_b, dst_b):
        def _fg(g, cc):
            sl = pl.ds(g * 16, 16)
            v = comb_b[sl]
            srcg_b[sl] = (v & 16383) + toff
            dst_b[sl] = lax.shift_right_logical(v, 14)
            return cc
        lax.fori_loop(0, _EK // 16, _fg, 0)

    def _fire(q):
        pltpu.async_copy(h_hbm.at[srcg_bufs[q]], rows_bufs[q], gsems[q])
        pltpu.async_copy(scores2_hbm.at[srcg_bufs[q]], ssv_bufs[q], gsems[q])
        pltpu.async_copy(scores2_hbm.at[dst_bufs[q]], sdv_bufs[q], gsems[q])

    plsc.subcore_barrier()

    for q in range(ndeep):
        pltpu.sync_copy(comb_hbm.at[cbase + q], comb_bufs[q])
        _fill_srcg(comb_bufs[q], srcg_bufs[q], dst_bufs[q])
        _fire(q)
        pltpu.async_copy(comb_hbm.at[cbase + q + ndeep], comb_bufs[q],
                         csems[q])

    def _dummy_wait(dst_ref, sem):
        pltpu.make_async_copy(h_hbm.at[pl.ds(0, _EK)], dst_ref, sem).wait()

    def _process(c, q, w):
        rows_b = rows_bufs[q]
        dst_b = dst_bufs[q]
        wr_w = wr_bufs[w]
        pdst_w = pdst_bufs[w]
        _dummy_wait(rows_b, gsems[q])
        pltpu.make_async_copy(scores2_hbm.at[pl.ds(0, _EK)], ssv_bufs[q],
                              gsems[q]).wait()
        pltpu.make_async_copy(scores2_hbm.at[pl.ds(0, _EK)], sdv_bufs[q],
                              gsems[q]).wait()
        pltpu.async_copy(rows_b, q_acc.at[dst_b], qsems[q], add=True)
        @pl.when(c >= 2)
        def _():
            _dummy_wait(wr_w, psems[w])

        def _grp(g, cc):
            sl = pl.ds(g * 16, 16)
            pdst_w[sl] = dst_b[sl]
            ss = ssv_bufs[q][sl]
            sd = sdv_bufs[q][sl]
            al = 1.0 / (1.0 + jnp.exp(sd - ss))
            for e in range(16):
                a = al[e]
                j = g * 16 + e
                for t in range(_H // 16):
                    sl2 = pl.ds(t * 16, 16)
                    wr_w[j, sl2] = rows_b[j, sl2] * a
            return cc
        lax.fori_loop(0, _EK // 16, _grp, 0)

        pltpu.async_copy(wr_w, p_acc.at[pdst_w], psems[w], add=True)
        _dummy_wait(rows_b, qsems[q])

        @pl.when(c + ndeep < nch)
        def _():
            pltpu.make_async_copy(comb_hbm.at[cbase], comb_bufs[q],
                                  csems[q]).wait()
            _fill_srcg(comb_bufs[q], srcg_bufs[q], dst_b)
            _fire(q)

            @pl.when(c + 2 * ndeep < nch)
            def _():
                pltpu.async_copy(comb_hbm.at[cbase + c + 2 * ndeep],
                                 comb_bufs[q], csems[q])

    def _six(i6, c):
        for k in range(6):
            _process(i6 * 6 + k, k % ndeep, k % 2)
        return c
    nmain = (nch // 6) * 6
    lax.fori_loop(0, nch // 6, _six, 0)
    for r in range(nch - nmain):
        c = nmain + r
        _process(jnp.int32(c), c % ndeep, c % 2)
    _dummy_wait(wr_bufs[nch % 2], psems[nch % 2])
    _dummy_wait(wr_bufs[(nch + 1) % 2], psems[(nch + 1) % 2])

    plsc.subcore_barrier()

    obase = sid * rows_per_tile
    nvalid = _N - (_NS - 1) * rows_per_tile

    @pl.when(sid < _NS - 1)
    def _():
        pltpu.sync_copy(p_acc.at[pl.ds(obase, rows_per_tile)],
                        p_out.at[cid, pl.ds(obase, rows_per_tile)])
        pltpu.sync_copy(q_acc.at[pl.ds(obase, rows_per_tile)],
                        q_out.at[cid, pl.ds(obase, rows_per_tile)])

    @pl.when(sid == _NS - 1)
    def _():
        pltpu.sync_copy(p_acc.at[pl.ds(obase, nvalid)],
                        p_out.at[cid, pl.ds(obase, nvalid)])
        pltpu.sync_copy(q_acc.at[pl.ds(obase, nvalid)],
                        q_out.at[cid, pl.ds(obase, nvalid)])


def _sc_edge(hsplit_flat, comb2d, scores2):
    mesh = plsc.VectorSubcoreMesh(core_axis_name="c", subcore_axis_name="s")
    f32 = jnp.float32
    i32 = jnp.int32
    ndeep = 3
    scratch = []
    for _ in range(ndeep):
        scratch.append(pltpu.VMEM((_EK,), i32))
    for _ in range(ndeep):
        scratch.append(pltpu.VMEM((_EK,), i32))
    for _ in range(ndeep):
        scratch.append(pltpu.VMEM((_EK,), i32))
    for _ in range(ndeep):
        scratch.append(pltpu.VMEM((_EK,), f32))
    for _ in range(ndeep):
        scratch.append(pltpu.VMEM((_EK,), f32))
    for _ in range(ndeep):
        scratch.append(pltpu.VMEM((_EK, _H), f32))
    scratch.append(pltpu.VMEM((_EK, _H), f32))
    scratch.append(pltpu.VMEM((_EK, _H), f32))
    scratch.append(pltpu.VMEM((_EK,), i32))
    scratch.append(pltpu.VMEM((_EK,), i32))
    scratch.append(pltpu.VMEM_SHARED((_NPAD, _H), f32))
    scratch.append(pltpu.VMEM_SHARED((_NPAD, _H), f32))
    for _ in range(3 * ndeep + 2):
        scratch.append(pltpu.SemaphoreType.DMA)
    run = pl.kernel(
        _sc_edge_body,
        out_type=(jax.ShapeDtypeStruct((_NC, _N, _H), f32),
                  jax.ShapeDtypeStruct((_NC, _N, _H), f32)),
        mesh=mesh,
        scratch_types=scratch,
        compiler_params=pltpu.CompilerParams(needs_layout_passes=False,
                                             use_tc_tiling_on_sc=False),
    )
    return run(hsplit_flat, comb2d, scores2)


def _fuse_body(self_w, fwd_w, bwd_w, comb_w, self_b, comb_b,
               ws_o, wd_o, wb_o, bias_o):
    c1 = comb_w[0, :_D, :]
    c2 = comb_w[0, _D:2 * _D, :]
    c3 = comb_w[0, 2 * _D:, :]
    f32 = jnp.float32
    ws_o[0] = jnp.dot(self_w[0], c1, preferred_element_type=f32)
    wf = jnp.dot(fwd_w[0], c2, preferred_element_type=f32)
    wb = jnp.dot(bwd_w[0], c3, preferred_element_type=f32)
    wd_o[0] = wf - wb
    wb_o[0] = wb
    bias_o[0, 0] = jnp.dot(self_b[0, 0], c1, preferred_element_type=f32) \
        + comb_b[0, 0]


def _fuse_weights(self_W, fwd_W, bwd_W, comb_W, self_b, comb_b):
    f32 = jnp.float32
    w_spec = pl.BlockSpec((1, _D, _D), lambda l: (l, 0, 0))
    b_spec = pl.BlockSpec((1, 1, _D), lambda l: (l, 0, 0))
    return pl.pallas_call(
        _fuse_body,
        grid=(_NL,),
        in_specs=[w_spec, w_spec, w_spec,
                  pl.BlockSpec((1, 3 * _D, _D), lambda l: (l, 0, 0)),
                  b_spec, b_spec],
        out_specs=[w_spec, w_spec, w_spec, b_spec],
        out_shape=[jax.ShapeDtypeStruct((_NL, _D, _D), f32),
                   jax.ShapeDtypeStruct((_NL, _D, _D), f32),
                   jax.ShapeDtypeStruct((_NL, _D, _D), f32),
                   jax.ShapeDtypeStruct((_NL, 1, _D), f32)],
    )(self_W, fwd_W, bwd_W, comb_W,
      self_b.reshape(_NL, 1, _D), comb_b.reshape(_NL, 1, _D))


def _prologue_body(x_ref, w_ref, b_ref, sw_ref, hs_o, sc_o):
    f32 = jnp.float32
    h = jnp.dot(x_ref[...], w_ref[...], preferred_element_type=f32)
    h = jax.nn.relu(h + b_ref[0])
    hs_o[0] = h[:, :_H]
    hs_o[1] = h[:, _H:]
    sc_o[0, 0] = jnp.dot(h, sw_ref[0, 0], preferred_element_type=f32)


def _prologue(x, emb_W, emb_b, sw0):
    f32 = jnp.float32
    return pl.pallas_call(
        _prologue_body,
        grid=(_NSTEPS,),
        in_specs=[pl.BlockSpec((_BN, _D), lambda i: (i, 0)),
                  pl.BlockSpec((_D, _D), lambda i: (0, 0)),
                  pl.BlockSpec((1, _D), lambda i: (0, 0)),
                  pl.BlockSpec((1, 1, _D), lambda i: (0, 0, 0))],
        out_specs=[pl.BlockSpec((2, _BN, _H), lambda i: (0, i, 0)),
                   pl.BlockSpec((1, 1, _BN), lambda i: (i, 0, 0))],
        out_shape=[jax.ShapeDtypeStruct((2, _N, _H), f32),
                   jax.ShapeDtypeStruct((_NSTEPS, 1, _BN), f32)],
    )(x, emb_W, emb_b.reshape(1, _D), sw0)


def _ln_relu_res(acc, h, g_ref, b_ref):
    mu = jnp.mean(acc, axis=-1, keepdims=True)
    var = jnp.mean((acc - mu) ** 2, axis=-1, keepdims=True)
    nrm = (acc - mu) / jnp.sqrt(var + 1e-5) * g_ref[0] + b_ref[0]
    return jax.nn.relu(nrm) + h


def _layer_body(hl, hr, pl_r, pr_r, ql, qr, ws, wd, wb, bias, g_ref, b_ref,
                sw_ref, hs_o, sc_o):
    f32 = jnp.float32
    h = jnp.concatenate([hl[0], hr[0]], axis=-1)
    p = jnp.concatenate([pl_r[0], pr_r[0]], axis=-1)
    q = jnp.concatenate([ql[0], qr[0]], axis=-1)
    acc = (jnp.dot(h, ws[...], preferred_element_type=f32)
           + jnp.dot(p, wd[...], preferred_element_type=f32)
           + jnp.dot(q, wb[...], preferred_element_type=f32)
           + bias[0])
    out = _ln_relu_res(acc, h, g_ref, b_ref)
    hs_o[0] = out[:, :_H]
    hs_o[1] = out[:, _H:]
    sc_o[0, 0] = jnp.dot(out, sw_ref[0, 0], preferred_element_type=f32)


def _layer(hs, p2, q2, ws, wd, wb, bias, ln_g1, ln_b1, sw_next):
    f32 = jnp.float32
    half = lambda j: pl.BlockSpec((1, _BN, _H), lambda i, j=j: (j, i, 0))
    wspec = pl.BlockSpec((_D, _D), lambda i: (0, 0))
    vspec = pl.BlockSpec((1, _D), lambda i: (0, 0))
    return pl.pallas_call(
        _layer_body,
        grid=(_NSTEPS,),
        in_specs=[half(0), half(1), half(0), half(1), half(0), half(1),
                  wspec, wspec, wspec, vspec, vspec, vspec,
                  pl.BlockSpec((1, 1, _D), lambda i: (0, 0, 0))],
        out_specs=[pl.BlockSpec((2, _BN, _H), lambda i: (0, i, 0)),
                   pl.BlockSpec((1, 1, _BN), lambda i: (i, 0, 0))],
        out_shape=[jax.ShapeDtypeStruct((2, _N, _H), f32),
                   jax.ShapeDtypeStruct((_NSTEPS, 1, _BN), f32)],
    )(hs, hs, p2, p2, q2, q2, ws, wd, wb, bias, ln_g1, ln_b1, sw_next)


def _final_body(hl, hr, pl_r, pr_r, ql, qr, ws, wd, wb, bias, g_ref, b_ref,
                batch_ref, pooled_o, sum_acc, cnt_acc):
    f32 = jnp.float32
    i = pl.program_id(0)
    h = jnp.concatenate([hl[0], hr[0]], axis=-1)
    p = jnp.concatenate([pl_r[0], pr_r[0]], axis=-1)
    q = jnp.concatenate([ql[0], qr[0]], axis=-1)
    acc = (jnp.dot(h, ws[...], preferred_element_type=f32)
           + jnp.dot(p, wd[...], preferred_element_type=f32)
           + jnp.dot(q, wb[...], preferred_element_type=f32)
           + bias[0])
    out = _ln_relu_res(acc, h, g_ref, b_ref)

    @pl.when(i == 0)
    def _():
        sum_acc[...] = jnp.zeros((_G, _D), f32)
        cnt_acc[...] = jnp.zeros((_G, 1), f32)

    gids = lax.broadcasted_iota(jnp.int32, (_G, _BN), 0)
    oh = (gids == batch_ref[0, 0][None, :]).astype(f32)
    sum_acc[...] += jnp.dot(oh, out, preferred_element_type=f32)
    cnt_acc[...] += jnp.sum(oh, axis=1, keepdims=True)

    @pl.when(i == _NSTEPS - 1)
    def _():
        pooled_o[...] = sum_acc[...] / jnp.maximum(cnt_acc[...], 1.0)


def _final_layer(hs, p2, q2, ws, wd, wb, bias, ln_g1, ln_b1, batch3d):
    f32 = jnp.float32
    half = lambda j: pl.BlockSpec((1, _BN, _H), lambda i, j=j: (j, i, 0))
    wspec = pl.BlockSpec((_D, _D), lambda i: (0, 0))
    vspec = pl.BlockSpec((1, _D), lambda i: (0, 0))
    return pl.pallas_call(
        _final_body,
        grid=(_NSTEPS,),
        in_specs=[half(0), half(1), half(0), half(1), half(0), half(1),
                  wspec, wspec, wspec, vspec, vspec, vspec,
                  pl.BlockSpec((1, 1, _BN), lambda i: (i, 0, 0))],
        out_specs=pl.BlockSpec((_G, _D), lambda i: (0, 0)),
        out_shape=jax.ShapeDtypeStruct((_G, _D), f32),
        scratch_shapes=[pltpu.VMEM((_G, _D), f32),
                        pltpu.VMEM((_G, 1), f32)],
    )(hs, hs, p2, p2, q2, q2, ws, wd, wb, bias, ln_g1, ln_b1, batch3d)


def kernel(x, edge_index, batch, emb_W, emb_b, score_W, score_b, fwd_W,
           bwd_W, self_W, self_b, comb_W, comb_b, ln_g, ln_b):
    src = edge_index[0].astype(jnp.int32)
    dst = edge_index[1].astype(jnp.int32)
    npad = _EPAD - _E
    comb = src + dst * 16384
    comb2d = jnp.concatenate(
        [comb, jnp.full((npad,), _N * 16384, jnp.int32)]).reshape(-1, _EK)
    batch3d = batch.astype(jnp.int32).reshape(_NSTEPS, 1, _BN)
    sw = score_W.reshape(_NL, 1, 1, _D)

    ws_s, wd_s, wb_s, bias_s = _fuse_weights(self_W, fwd_W, bwd_W, comb_W,
                                             self_b, comb_b)

    hs, sc = _prologue(x, emb_W, emb_b, sw[0])
    for l in range(_NL):
        sflat = sc.reshape(_N)
        scores2 = jnp.concatenate(
            [sflat, sflat, jnp.zeros((2 * _NPAD - 2 * _N,), jnp.float32)])
        p2, q2 = _sc_edge(hs.reshape(_NC * _N, _H), comb2d, scores2)
        args = (hs, p2, q2, ws_s[l], wd_s[l], wb_s[l], bias_s[l],
                ln_g[l].reshape(1, _D), ln_b[l].reshape(1, _D))
        if l < _NL - 1:
            hs, sc = _layer(*args, sw[l + 1])
        else:
            pooled = _final_layer(*args, batch3d)
    return (pooled, 0)


# speedup vs baseline: 16.8788x; 1.3280x over previous
import jax
import jax.numpy as jnp
from jax import lax
from jax.experimental import pallas as pl
from jax.experimental.pallas import tpu as pltpu
from jax.experimental.pallas import tpu_sc as plsc

_N = 10000
_E = 320000
_D = 128
_H = 64
_NL = 3
_G = 64
_NC = 2
_NS = 16
_NPAD = 10240
_EPAD = 327680
_EPS = _EPAD // _NS
_EK = 128
_BN = 1000
_NSTEPS = _N // _BN


def _sc_edge_body(h_hbm, comb_hbm, scores2_hbm, p_out, q_out,
                  comb0, comb1, comb2,
                  srcg0, srcg1, srcg2,
                  dst0, dst1, dst2,
                  ssv0, ssv1, ssv2,
                  sdv0, sdv1, sdv2,
                  rows0, rows1, rows2, wra, wrb, pdst0, pdst1,
                  p_acc, q_acc,
                  gsem0, gsem1, gsem2,
                  csem0, csem1, csem2,
                  qsem0, qsem1, qsem2,
                  psem0, psem1):
    cid = lax.axis_index("c")
    sid = lax.axis_index("s")
    nch = _EPS // _EK
    comb_bufs = (comb0, comb1, comb2)
    srcg_bufs = (srcg0, srcg1, srcg2)
    dst_bufs = (dst0, dst1, dst2)
    ssv_bufs = (ssv0, ssv1, ssv2)
    sdv_bufs = (sdv0, sdv1, sdv2)
    rows_bufs = (rows0, rows1, rows2)
    wr_bufs = (wra, wrb)
    pdst_bufs = (pdst0, pdst1)
    gsems = (gsem0, gsem1, gsem2)
    csems = (csem0, csem1, csem2)
    qsems = (qsem0, qsem1, qsem2)
    psems = (psem0, psem1)
    ndeep = 3

    def _zrow(j, c):
        for t in range(_H // 16):
            wra[j, pl.ds(t * 16, 16)] = jnp.zeros((16,), jnp.float32)
        return c
    lax.fori_loop(0, _EK, _zrow, 0)

    rows_per_tile = _NPAD // _NS
    def _zacc(ci, c):
        base = sid * rows_per_tile + ci * _EK
        pltpu.sync_copy(wra, p_acc.at[pl.ds(base, _EK)])
        pltpu.sync_copy(wra, q_acc.at[pl.ds(base, _EK)])
        return c
    lax.fori_loop(0, rows_per_tile // _EK, _zacc, 0)

    cbase = sid * nch
    toff = cid * _N

    def _fill_srcg(comb_b, srcg_b, dst_b):
        def _fg(g, cc):
            sl = pl.ds(g * 16, 16)
            v = comb_b[sl]
            srcg_b[sl] = (v & 16383) + toff
            dst_b[sl] = lax.shift_right_logical(v, 14)
            return cc
        lax.fori_loop(0, _EK // 16, _fg, 0)

    def _fire(q):
        pltpu.async_copy(h_hbm.at[srcg_bufs[q]], rows_bufs[q], gsems[q])
        pltpu.async_copy(scores2_hbm.at[srcg_bufs[q]], ssv_bufs[q], gsems[q])
        pltpu.async_copy(scores2_hbm.at[dst_bufs[q]], sdv_bufs[q], gsems[q])

    plsc.subcore_barrier()

    for q in range(ndeep):
        pltpu.sync_copy(comb_hbm.at[cbase + q], comb_bufs[q])
        _fill_srcg(comb_bufs[q], srcg_bufs[q], dst_bufs[q])
        _fire(q)
        pltpu.async_copy(comb_hbm.at[cbase + q + ndeep], comb_bufs[q],
                         csems[q])

    def _dummy_wait(dst_ref, sem):
        pltpu.make_async_copy(h_hbm.at[pl.ds(0, _EK)], dst_ref, sem).wait()

    def _process(c, q, w):
        rows_b = rows_bufs[q]
        dst_b = dst_bufs[q]
        wr_w = wr_bufs[w]
        pdst_w = pdst_bufs[w]
        _dummy_wait(rows_b, gsems[q])
        pltpu.make_async_copy(scores2_hbm.at[pl.ds(0, _EK)], ssv_bufs[q],
                              gsems[q]).wait()
        pltpu.make_async_copy(scores2_hbm.at[pl.ds(0, _EK)], sdv_bufs[q],
                              gsems[q]).wait()
        pltpu.async_copy(rows_b, q_acc.at[dst_b], qsems[q], add=True)
        @pl.when(c >= 2)
        def _():
            _dummy_wait(wr_w, psems[w])

        @plsc.parallel_loop(0, _EK // 16, 1, unroll=2)
        def _grp(g):
            sl = pl.ds(g * 16, 16)
            pdst_w[sl] = dst_b[sl]
            ss = ssv_bufs[q][sl]
            sd = sdv_bufs[q][sl]
            al = 1.0 / (1.0 + jnp.exp(sd - ss))
            for e in range(16):
                a = al[e]
                j = g * 16 + e
                for t in range(_H // 16):
                    sl2 = pl.ds(t * 16, 16)
                    wr_w[j, sl2] = rows_b[j, sl2] * a

        pltpu.async_copy(wr_w, p_acc.at[pdst_w], psems[w], add=True)
        _dummy_wait(rows_b, qsems[q])

        @pl.when(c + ndeep < nch)
        def _():
            pltpu.make_async_copy(comb_hbm.at[cbase], comb_bufs[q],
                                  csems[q]).wait()
            _fill_srcg(comb_bufs[q], srcg_bufs[q], dst_b)
            _fire(q)

            @pl.when(c + 2 * ndeep < nch)
            def _():
                pltpu.async_copy(comb_hbm.at[cbase + c + 2 * ndeep],
                                 comb_bufs[q], csems[q])

    def _six(i6, c):
        for k in range(6):
            _process(i6 * 6 + k, k % ndeep, k % 2)
        return c
    nmain = (nch // 6) * 6
    lax.fori_loop(0, nch // 6, _six, 0)
    for r in range(nch - nmain):
        c = nmain + r
        _process(jnp.int32(c), c % ndeep, c % 2)
    _dummy_wait(wr_bufs[nch % 2], psems[nch % 2])
    _dummy_wait(wr_bufs[(nch + 1) % 2], psems[(nch + 1) % 2])

    plsc.subcore_barrier()

    obase = sid * rows_per_tile
    nvalid = _N - (_NS - 1) * rows_per_tile

    @pl.when(sid < _NS - 1)
    def _():
        pltpu.sync_copy(p_acc.at[pl.ds(obase, rows_per_tile)],
                        p_out.at[cid, pl.ds(obase, rows_per_tile)])
        pltpu.sync_copy(q_acc.at[pl.ds(obase, rows_per_tile)],
                        q_out.at[cid, pl.ds(obase, rows_per_tile)])

    @pl.when(sid == _NS - 1)
    def _():
        pltpu.sync_copy(p_acc.at[pl.ds(obase, nvalid)],
                        p_out.at[cid, pl.ds(obase, nvalid)])
        pltpu.sync_copy(q_acc.at[pl.ds(obase, nvalid)],
                        q_out.at[cid, pl.ds(obase, nvalid)])


def _sc_edge(hsplit_flat, comb2d, scores2):
    mesh = plsc.VectorSubcoreMesh(core_axis_name="c", subcore_axis_name="s")
    f32 = jnp.float32
    i32 = jnp.int32
    ndeep = 3
    scratch = []
    for _ in range(ndeep):
        scratch.append(pltpu.VMEM((_EK,), i32))
    for _ in range(ndeep):
        scratch.append(pltpu.VMEM((_EK,), i32))
    for _ in range(ndeep):
        scratch.append(pltpu.VMEM((_EK,), i32))
    for _ in range(ndeep):
        scratch.append(pltpu.VMEM((_EK,), f32))
    for _ in range(ndeep):
        scratch.append(pltpu.VMEM((_EK,), f32))
    for _ in range(ndeep):
        scratch.append(pltpu.VMEM((_EK, _H), f32))
    scratch.append(pltpu.VMEM((_EK, _H), f32))
    scratch.append(pltpu.VMEM((_EK, _H), f32))
    scratch.append(pltpu.VMEM((_EK,), i32))
    scratch.append(pltpu.VMEM((_EK,), i32))
    scratch.append(pltpu.VMEM_SHARED((_NPAD, _H), f32))
    scratch.append(pltpu.VMEM_SHARED((_NPAD, _H), f32))
    for _ in range(3 * ndeep + 2):
        scratch.append(pltpu.SemaphoreType.DMA)
    run = pl.kernel(
        _sc_edge_body,
        out_type=(jax.ShapeDtypeStruct((_NC, _N, _H), f32),
                  jax.ShapeDtypeStruct((_NC, _N, _H), f32)),
        mesh=mesh,
        scratch_types=scratch,
        compiler_params=pltpu.CompilerParams(needs_layout_passes=False,
                                             use_tc_tiling_on_sc=False),
    )
    return run(hsplit_flat, comb2d, scores2)


def _fuse_body(self_w, fwd_w, bwd_w, comb_w, self_b, comb_b,
               ws_o, wd_o, wb_o, bias_o):
    c1 = comb_w[0, :_D, :]
    c2 = comb_w[0, _D:2 * _D, :]
    c3 = comb_w[0, 2 * _D:, :]
    f32 = jnp.float32
    ws_o[0] = jnp.dot(self_w[0], c1, preferred_element_type=f32)
    wf = jnp.dot(fwd_w[0], c2, preferred_element_type=f32)
    wb = jnp.dot(bwd_w[0], c3, preferred_element_type=f32)
    wd_o[0] = wf - wb
    wb_o[0] = wb
    bias_o[0, 0] = jnp.dot(self_b[0, 0], c1, preferred_element_type=f32) \
        + comb_b[0, 0]


def _fuse_weights(self_W, fwd_W, bwd_W, comb_W, self_b, comb_b):
    f32 = jnp.float32
    w_spec = pl.BlockSpec((1, _D, _D), lambda l: (l, 0, 0))
    b_spec = pl.BlockSpec((1, 1, _D), lambda l: (l, 0, 0))
    return pl.pallas_call(
        _fuse_body,
        grid=(_NL,),
        in_specs=[w_spec, w_spec, w_spec,
                  pl.BlockSpec((1, 3 * _D, _D), lambda l: (l, 0, 0)),
                  b_spec, b_spec],
        out_specs=[w_spec, w_spec, w_spec, b_spec],
        out_shape=[jax.ShapeDtypeStruct((_NL, _D, _D), f32),
                   jax.ShapeDtypeStruct((_NL, _D, _D), f32),
                   jax.ShapeDtypeStruct((_NL, _D, _D), f32),
                   jax.ShapeDtypeStruct((_NL, 1, _D), f32)],
    )(self_W, fwd_W, bwd_W, comb_W,
      self_b.reshape(_NL, 1, _D), comb_b.reshape(_NL, 1, _D))


def _prologue_body(x_ref, w_ref, b_ref, sw_ref, hs_o, sc_o):
    f32 = jnp.float32
    h = jnp.dot(x_ref[...], w_ref[...], preferred_element_type=f32)
    h = jax.nn.relu(h + b_ref[0])
    hs_o[0] = h[:, :_H]
    hs_o[1] = h[:, _H:]
    sc_o[0, 0] = jnp.dot(h, sw_ref[0, 0], preferred_element_type=f32)


def _prologue(x, emb_W, emb_b, sw0):
    f32 = jnp.float32
    return pl.pallas_call(
        _prologue_body,
        grid=(_NSTEPS,),
        in_specs=[pl.BlockSpec((_BN, _D), lambda i: (i, 0)),
                  pl.BlockSpec((_D, _D), lambda i: (0, 0)),
                  pl.BlockSpec((1, _D), lambda i: (0, 0)),
                  pl.BlockSpec((1, 1, _D), lambda i: (0, 0, 0))],
        out_specs=[pl.BlockSpec((2, _BN, _H), lambda i: (0, i, 0)),
                   pl.BlockSpec((1, 1, _BN), lambda i: (i, 0, 0))],
        out_shape=[jax.ShapeDtypeStruct((2, _N, _H), f32),
                   jax.ShapeDtypeStruct((_NSTEPS, 1, _BN), f32)],
    )(x, emb_W, emb_b.reshape(1, _D), sw0)


def _ln_relu_res(acc, h, g_ref, b_ref):
    mu = jnp.mean(acc, axis=-1, keepdims=True)
    var = jnp.mean((acc - mu) ** 2, axis=-1, keepdims=True)
    nrm = (acc - mu) / jnp.sqrt(var + 1e-5) * g_ref[0] + b_ref[0]
    return jax.nn.relu(nrm) + h


def _layer_body(hl, hr, pl_r, pr_r, ql, qr, ws, wd, wb, bias, g_ref, b_ref,
                sw_ref, hs_o, sc_o):
    f32 = jnp.float32
    h = jnp.concatenate([hl[0], hr[0]], axis=-1)
    p = jnp.concatenate([pl_r[0], pr_r[0]], axis=-1)
    q = jnp.concatenate([ql[0], qr[0]], axis=-1)
    acc = (jnp.dot(h, ws[...], preferred_element_type=f32)
           + jnp.dot(p, wd[...], preferred_element_type=f32)
           + jnp.dot(q, wb[...], preferred_element_type=f32)
           + bias[0])
    out = _ln_relu_res(acc, h, g_ref, b_ref)
    hs_o[0] = out[:, :_H]
    hs_o[1] = out[:, _H:]
    sc_o[0, 0] = jnp.dot(out, sw_ref[0, 0], preferred_element_type=f32)


def _layer(hs, p2, q2, ws, wd, wb, bias, ln_g1, ln_b1, sw_next):
    f32 = jnp.float32
    half = lambda j: pl.BlockSpec((1, _BN, _H), lambda i, j=j: (j, i, 0))
    wspec = pl.BlockSpec((_D, _D), lambda i: (0, 0))
    vspec = pl.BlockSpec((1, _D), lambda i: (0, 0))
    return pl.pallas_call(
        _layer_body,
        grid=(_NSTEPS,),
        in_specs=[half(0), half(1), half(0), half(1), half(0), half(1),
                  wspec, wspec, wspec, vspec, vspec, vspec,
                  pl.BlockSpec((1, 1, _D), lambda i: (0, 0, 0))],
        out_specs=[pl.BlockSpec((2, _BN, _H), lambda i: (0, i, 0)),
                   pl.BlockSpec((1, 1, _BN), lambda i: (i, 0, 0))],
        out_shape=[jax.ShapeDtypeStruct((2, _N, _H), f32),
                   jax.ShapeDtypeStruct((_NSTEPS, 1, _BN), f32)],
    )(hs, hs, p2, p2, q2, q2, ws, wd, wb, bias, ln_g1, ln_b1, sw_next)


def _final_body(hl, hr, pl_r, pr_r, ql, qr, ws, wd, wb, bias, g_ref, b_ref,
                batch_ref, pooled_o, sum_acc, cnt_acc):
    f32 = jnp.float32
    i = pl.program_id(0)
    h = jnp.concatenate([hl[0], hr[0]], axis=-1)
    p = jnp.concatenate([pl_r[0], pr_r[0]], axis=-1)
    q = jnp.concatenate([ql[0], qr[0]], axis=-1)
    acc = (jnp.dot(h, ws[...], preferred_element_type=f32)
           + jnp.dot(p, wd[...], preferred_element_type=f32)
           + jnp.dot(q, wb[...], preferred_element_type=f32)
           + bias[0])
    out = _ln_relu_res(acc, h, g_ref, b_ref)

    @pl.when(i == 0)
    def _():
        sum_acc[...] = jnp.zeros((_G, _D), f32)
        cnt_acc[...] = jnp.zeros((_G, 1), f32)

    gids = lax.broadcasted_iota(jnp.int32, (_G, _BN), 0)
    oh = (gids == batch_ref[0, 0][None, :]).astype(f32)
    sum_acc[...] += jnp.dot(oh, out, preferred_element_type=f32)
    cnt_acc[...] += jnp.sum(oh, axis=1, keepdims=True)

    @pl.when(i == _NSTEPS - 1)
    def _():
        pooled_o[...] = sum_acc[...] / jnp.maximum(cnt_acc[...], 1.0)


def _final_layer(hs, p2, q2, ws, wd, wb, bias, ln_g1, ln_b1, batch3d):
    f32 = jnp.float32
    half = lambda j: pl.BlockSpec((1, _BN, _H), lambda i, j=j: (j, i, 0))
    wspec = pl.BlockSpec((_D, _D), lambda i: (0, 0))
    vspec = pl.BlockSpec((1, _D), lambda i: (0, 0))
    return pl.pallas_call(
        _final_body,
        grid=(_NSTEPS,),
        in_specs=[half(0), half(1), half(0), half(1), half(0), half(1),
                  wspec, wspec, wspec, vspec, vspec, vspec,
                  pl.BlockSpec((1, 1, _BN), lambda i: (i, 0, 0))],
        out_specs=pl.BlockSpec((_G, _D), lambda i: (0, 0)),
        out_shape=jax.ShapeDtypeStruct((_G, _D), f32),
        scratch_shapes=[pltpu.VMEM((_G, _D), f32),
                        pltpu.VMEM((_G, 1), f32)],
    )(hs, hs, p2, p2, q2, q2, ws, wd, wb, bias, ln_g1, ln_b1, batch3d)


def kernel(x, edge_index, batch, emb_W, emb_b, score_W, score_b, fwd_W,
           bwd_W, self_W, self_b, comb_W, comb_b, ln_g, ln_b):
    src = edge_index[0].astype(jnp.int32)
    dst = edge_index[1].astype(jnp.int32)
    npad = _EPAD - _E
    comb = src + dst * 16384
    comb2d = jnp.concatenate(
        [comb, jnp.full((npad,), _N * 16384, jnp.int32)]).reshape(-1, _EK)
    batch3d = batch.astype(jnp.int32).reshape(_NSTEPS, 1, _BN)
    sw = score_W.reshape(_NL, 1, 1, _D)

    ws_s, wd_s, wb_s, bias_s = _fuse_weights(self_W, fwd_W, bwd_W, comb_W,
                                             self_b, comb_b)

    hs, sc = _prologue(x, emb_W, emb_b, sw[0])
    for l in range(_NL):
        sflat = sc.reshape(_N)
        scores2 = jnp.concatenate(
            [sflat, sflat, jnp.zeros((2 * _NPAD - 2 * _N,), jnp.float32)])
        p2, q2 = _sc_edge(hs.reshape(_NC * _N, _H), comb2d, scores2)
        args = (hs, p2, q2, ws_s[l], wd_s[l], wb_s[l], bias_s[l],
                ln_g[l].reshape(1, _D), ln_b[l].reshape(1, _D))
        if l < _NL - 1:
            hs, sc = _layer(*args, sw[l + 1])
        else:
            pooled = _final_layer(*args, batch3d)
    return (pooled, 0)
